# Initial kernel scaffold; baseline (speedup 1.0000x reference)
import jax
import jax.numpy as jnp
from jax.experimental import pallas as pl


def kernel(atom_features, bond_features, motif_features, params, edge_index, motif_edge_index):
    raise NotImplementedError("write your pallas kernel here")


# speedup vs baseline: 5.3160x; 5.3160x over previous
import functools

import jax
import jax.numpy as jnp
from jax import lax
from jax.experimental import pallas as pl
from jax.experimental.pallas import tpu as pltpu
from jax.experimental.pallas import tpu_sc as plsc

N_ATOM = 2048
E_ATOM = 65536
N_MOTIF = 512
E_MOTIF = 2048
HIDDEN = 256
HEADS = 4
HD = HIDDEN // HEADS

NW = 32
CH = 32
EBLK = 4096
MCH = N_MOTIF // NW

RB = 256
NRB = N_ATOM // RB

_PREC = jax.lax.Precision.HIGHEST


@functools.cache
def _sc_adjacency_kernel():
    mesh = plsc.VectorSubcoreMesh(core_axis_name="c", subcore_axis_name="s")
    return pl.kernel(
        _sc_adjacency_body,
        mesh=mesh,
        compiler_params=pltpu.CompilerParams(needs_layout_passes=False),
        out_type=[
            jax.ShapeDtypeStruct((N_ATOM * N_ATOM,), jnp.float32),
            jax.ShapeDtypeStruct((N_MOTIF * N_MOTIF,), jnp.float32),
        ],
        scratch_types=[
            pltpu.VMEM((CH * N_ATOM,), jnp.float32),
            pltpu.VMEM((EBLK,), jnp.int32),
            pltpu.VMEM((EBLK,), jnp.int32),
            pltpu.VMEM((MCH * N_MOTIF,), jnp.float32),
            pltpu.VMEM((E_MOTIF,), jnp.int32),
            pltpu.VMEM((E_MOTIF,), jnp.int32),
        ],
    )


def _sc_adjacency_body(src_hbm, dst_hbm, msrc_hbm, mdst_hbm, a_out, am_out,
                       buf, sbuf, dbuf, mbuf, msbuf, mdbuf):
    wid = lax.axis_index("s") * 2 + lax.axis_index("c")
    ones = jnp.ones((16,), jnp.float32)
    zeros = jnp.zeros((16,), jnp.float32)

    for chunk_i in range(N_ATOM // CH // NW):
        chunk = wid * (N_ATOM // CH // NW) + chunk_i
        base = chunk * CH

        def zbody(i, _):
            for k in range(8):
                buf[pl.ds(i * 128 + k * 16, 16)] = zeros
            return _
        lax.fori_loop(0, CH * N_ATOM // 128, zbody, 0)

        for blk in range(E_ATOM // EBLK):
            pltpu.sync_copy(src_hbm.at[pl.ds(blk * EBLK, EBLK)], sbuf)
            pltpu.sync_copy(dst_hbm.at[pl.ds(blk * EBLK, EBLK)], dbuf)

            def ebody(i, _):
                for k in range(4):
                    off = i * 64 + k * 16
                    s = sbuf[pl.ds(off, 16)]
                    dvec = dbuf[pl.ds(off, 16)]
                    rel = dvec - base
                    m = (rel >= 0) & (rel < CH)
                    idx = jnp.where(m, rel * N_ATOM + s, 0)
                    plsc.addupdate_scatter(buf, [idx], ones, mask=m)
                return _
            lax.fori_loop(0, EBLK // 64, ebody, 0)

        pltpu.sync_copy(buf, a_out.at[pl.ds(base * N_ATOM, CH * N_ATOM)])

    mbase = wid * MCH
    pltpu.sync_copy(msrc_hbm, msbuf)
    pltpu.sync_copy(mdst_hbm, mdbuf)

    def mzbody(i, _):
        for k in range(8):
            mbuf[pl.ds(i * 128 + k * 16, 16)] = zeros
        return _
    lax.fori_loop(0, MCH * N_MOTIF // 128, mzbody, 0)

    def mebody(i, _):
        for k in range(4):
            off = i * 64 + k * 16
            s = msbuf[pl.ds(off, 16)]
            dvec = mdbuf[pl.ds(off, 16)]
            rel = dvec - mbase
            m = (rel >= 0) & (rel < MCH)
            idx = jnp.where(m, rel * N_MOTIF + s, 0)
            plsc.addupdate_scatter(mbuf, [idx], ones, mask=m)
        return _
    lax.fori_loop(0, E_MOTIF // 64, mebody, 0)

    pltpu.sync_copy(mbuf, am_out.at[pl.ds(mbase * N_MOTIF, MCH * N_MOTIF)])


def _dot(a, b, ca=1, cb=0):
    return lax.dot_general(a, b, (((ca,), (cb,)), ((), ())), precision=_PREC)


def _full(shape):
    return pl.BlockSpec(shape, lambda *_: (0,) * len(shape))


def _dinv_body(a_ref, dinv_ref):
    deg = jnp.sum(a_ref[...], axis=1, keepdims=True) + 1.0
    dinv_ref[...] = lax.rsqrt(deg)


def _dinv(A):
    return pl.pallas_call(
        _dinv_body,
        grid=(NRB,),
        in_specs=[pl.BlockSpec((RB, N_ATOM), lambda i: (i, 0))],
        out_specs=pl.BlockSpec((RB, 1), lambda i: (i, 0)),
        out_shape=jax.ShapeDtypeStruct((N_ATOM, 1), jnp.float32),
    )(A)


def _embed_body(x_ref, w_ref, b_ref, o_ref):
    o_ref[...] = _dot(x_ref[...], w_ref[...]) + b_ref[...]


def _embed(x, w, b, rb):
    n, fin = x.shape
    fout = w.shape[1]
    return pl.pallas_call(
        _embed_body,
        grid=(n // rb,),
        in_specs=[pl.BlockSpec((rb, fin), lambda i: (i, 0)),
                  _full((fin, fout)), _full((1, fout))],
        out_specs=pl.BlockSpec((rb, fout), lambda i: (i, 0)),
        out_shape=jax.ShapeDtypeStruct((n, fout), jnp.float32),
    )(x, w, b.reshape(1, fout))


def _gcn_p1_body(x_ref, w_ref, dinv_ref, o_ref):
    o_ref[...] = dinv_ref[...] * _dot(x_ref[...], w_ref[...])


def _gcn_layer(A, x, dinv, w, b):
    hd = pl.pallas_call(
        _gcn_p1_body,
        grid=(NRB,),
        in_specs=[pl.BlockSpec((RB, HIDDEN), lambda i: (i, 0)),
                  _full((HIDDEN, HIDDEN)),
                  pl.BlockSpec((RB, 1), lambda i: (i, 0))],
        out_specs=pl.BlockSpec((RB, HIDDEN), lambda i: (i, 0)),
        out_shape=jax.ShapeDtypeStruct((N_ATOM, HIDDEN), jnp.float32),
    )(x, w, dinv)

    def p2(a_ref, hdf_ref, hdb_ref, dinv_ref, b_ref, o_ref):
        t = _dot(a_ref[...], hdf_ref[...]) + hdb_ref[...]
        o_ref[...] = jnp.maximum(dinv_ref[...] * t + b_ref[...], 0.0)

    return pl.pallas_call(
        p2,
        grid=(NRB,),
        in_specs=[pl.BlockSpec((RB, N_ATOM), lambda i: (i, 0)),
                  _full((N_ATOM, HIDDEN)),
                  pl.BlockSpec((RB, HIDDEN), lambda i: (i, 0)),
                  pl.BlockSpec((RB, 1), lambda i: (i, 0)),
                  _full((1, HIDDEN))],
        out_specs=pl.BlockSpec((RB, HIDDEN), lambda i: (i, 0)),
        out_shape=jax.ShapeDtypeStruct((N_ATOM, HIDDEN), jnp.float32),
    )(A, hd, hd, dinv, b.reshape(1, HIDDEN))


def _motif_body(am_ref, mf_ref, mw_ref, mb_ref,
                w1_ref, b1_ref, w2_ref, b2_ref, o_ref):
    m = _dot(mf_ref[...], mw_ref[...]) + mb_ref[...]
    Am = am_ref[...]
    for i in range(3):
        h = m + _dot(Am, m)
        h1 = jnp.maximum(_dot(h, w1_ref[i]) + b1_ref[i][None, :], 0.0)
        m = jnp.maximum(_dot(h1, w2_ref[i]) + b2_ref[i][None, :], 0.0)
    o_ref[...] = m


def _motif_stack(Am, motif_f, mw, mb, w1, b1, w2, b2):
    return pl.pallas_call(
        _motif_body,
        out_shape=jax.ShapeDtypeStruct((N_MOTIF, HIDDEN), jnp.float32),
    )(Am, motif_f, mw, mb.reshape(1, HIDDEN), w1, b1, w2, b2)


def _qkv_body(x_ref, w_ref, b_ref, o_ref):
    o_ref[0] = _dot(x_ref[...], w_ref[...], 1, 1) + b_ref[0]


def _qkv(x, wqkv, bqkv):
    L = x.shape[0]
    return pl.pallas_call(
        _qkv_body,
        grid=(3 * HEADS,),
        in_specs=[_full((L, HIDDEN)),
                  pl.BlockSpec((HD, HIDDEN), lambda h: (h, 0)),
                  pl.BlockSpec((1, 1, HD), lambda h: (h, 0, 0))],
        out_specs=pl.BlockSpec((1, L, HD), lambda h: (h, 0, 0)),
        out_shape=jax.ShapeDtypeStruct((3 * HEADS, L, HD), jnp.float32),
    )(x, wqkv, bqkv.reshape(3 * HEADS, 1, HD))


def _attn_pool_body(q_ref, k_ref, v_ref, o_ref):
    qb = pl.program_id(1)
    L = k_ref.shape[1]
    q = q_ref[0]
    k = k_ref[0]
    v = v_ref[0]
    s = lax.dot_general(q, k, (((1,), (1,)), ((), ())),
                        precision=_PREC) * (1.0 / float(HD) ** 0.5)
    s = s - jnp.max(s, axis=1, keepdims=True)
    e = jnp.exp(s)
    p = e / jnp.sum(e, axis=1, keepdims=True)
    o = _dot(p, v)
    colsum = jnp.sum(o, axis=0, keepdims=True) * (1.0 / float(L))

    @pl.when(qb == 0)
    def _():
        o_ref[...] = jnp.zeros_like(o_ref)

    o_ref[0] += colsum


def _attn_pool(qkv, L, qb):
    return pl.pallas_call(
        _attn_pool_body,
        grid=(HEADS, L // qb),
        in_specs=[
            pl.BlockSpec((1, qb, HD), lambda h, j: (h, j, 0)),
            pl.BlockSpec((1, L, HD), lambda h, j: (HEADS + h, 0, 0)),
            pl.BlockSpec((1, L, HD), lambda h, j: (2 * HEADS + h, 0, 0)),
        ],
        out_specs=pl.BlockSpec((1, 1, HD), lambda h, j: (h, 0, 0)),
        out_shape=jax.ShapeDtypeStruct((HEADS, 1, HD), jnp.float32),
    )(qkv, qkv, qkv)


def _final_body(am_ref, mm_ref, wo_ref, bo_ref,
                w1_ref, b1_ref, w2_ref, b2_ref, o_ref):
    ag = _dot(am_ref[...], wo_ref[...], 1, 1) + bo_ref[...]
    mg = _dot(mm_ref[...], wo_ref[...], 1, 1) + bo_ref[...]
    c = jnp.concatenate([ag, mg], axis=1)
    h = jnp.maximum(_dot(c, w1_ref[...]) + b1_ref[...], 0.0)
    o_ref[...] = _dot(h, w2_ref[...]) + b2_ref[...]


def _final(atom_mean, motif_mean, wo, bo, w1, b1, w2, b2):
    return pl.pallas_call(
        _final_body,
        out_shape=jax.ShapeDtypeStruct((1, 128), jnp.float32),
    )(atom_mean, motif_mean, wo, bo.reshape(1, HIDDEN),
      w1, b1.reshape(1, HIDDEN), w2, b2.reshape(1, 128))


def _build_adjacency(src, dst, msrc, mdst):
    a_flat, am_flat = _sc_adjacency_kernel()(src, dst, msrc, mdst)
    return (a_flat.reshape(N_ATOM, N_ATOM),
            am_flat.reshape(N_MOTIF, N_MOTIF))


def kernel(atom_features, bond_features, motif_features, params,
           edge_index, motif_edge_index):
    del bond_features
    p = params
    A, Am = _build_adjacency(edge_index[0], edge_index[1],
                             motif_edge_index[0], motif_edge_index[1])

    dinv = _dinv(A)
    x = _embed(atom_features, p['atom_W'], p['atom_b'], RB)
    for i in range(3):
        x = _gcn_layer(A, x, dinv, p['gcn_W'][i], p['gcn_b'][i])

    m = _motif_stack(Am, motif_features, p['motif_W'], p['motif_b'],
                     p['gin_W1'], p['gin_b1'], p['gin_W2'], p['gin_b2'])

    qkv_a = _qkv(x, p['attn_Wqkv'], p['attn_bqkv'])
    qkv_m = _qkv(m, p['attn_Wqkv'], p['attn_bqkv'])
    am = _attn_pool(qkv_a, N_ATOM, RB).reshape(1, HIDDEN)
    mm = _attn_pool(qkv_m, N_MOTIF, RB).reshape(1, HIDDEN)

    latent = _final(am, mm, p['attn_Wo'], p['attn_bo'],
                    p['proj_W1'], p['proj_b1'], p['proj_W2'], p['proj_b2'])
    return latent.reshape(128)


# speedup vs baseline: 8.9127x; 1.6766x over previous
import functools

import jax
import jax.numpy as jnp
from jax import lax
from jax.experimental import pallas as pl
from jax.experimental.pallas import tpu as pltpu
from jax.experimental.pallas import tpu_sc as plsc

N_ATOM = 2048
E_ATOM = 65536
N_MOTIF = 512
E_MOTIF = 2048
HIDDEN = 256
HEADS = 4
HD = HIDDEN // HEADS

NW = 32
CH = 32
EBLK = 4096
MCH = N_MOTIF // NW

RB = 256
NRB = N_ATOM // RB

_PREC = jax.lax.Precision.DEFAULT


@functools.cache
def _sc_adjacency_kernel():
    mesh = plsc.VectorSubcoreMesh(core_axis_name="c", subcore_axis_name="s")
    return pl.kernel(
        _sc_adjacency_body,
        mesh=mesh,
        compiler_params=pltpu.CompilerParams(needs_layout_passes=False),
        out_type=[
            jax.ShapeDtypeStruct((N_ATOM * N_ATOM,), jnp.float32),
            jax.ShapeDtypeStruct((N_MOTIF * N_MOTIF,), jnp.float32),
        ],
        scratch_types=[
            pltpu.VMEM((CH * N_ATOM,), jnp.float32),
            pltpu.VMEM((EBLK,), jnp.int32),
            pltpu.VMEM((EBLK,), jnp.int32),
            pltpu.VMEM((MCH * N_MOTIF,), jnp.float32),
            pltpu.VMEM((E_MOTIF,), jnp.int32),
            pltpu.VMEM((E_MOTIF,), jnp.int32),
        ],
    )


def _sc_adjacency_body(src_hbm, dst_hbm, msrc_hbm, mdst_hbm, a_out, am_out,
                       buf, sbuf, dbuf, mbuf, msbuf, mdbuf):
    wid = lax.axis_index("s") * 2 + lax.axis_index("c")
    ones = jnp.ones((16,), jnp.float32)
    zeros = jnp.zeros((16,), jnp.float32)

    for chunk_i in range(N_ATOM // CH // NW):
        chunk = wid * (N_ATOM // CH // NW) + chunk_i
        base = chunk * CH

        def zbody(i, _):
            for k in range(8):
                buf[pl.ds(i * 128 + k * 16, 16)] = zeros
            return _
        lax.fori_loop(0, CH * N_ATOM // 128, zbody, 0)

        for blk in range(E_ATOM // EBLK):
            pltpu.sync_copy(src_hbm.at[pl.ds(blk * EBLK, EBLK)], sbuf)
            pltpu.sync_copy(dst_hbm.at[pl.ds(blk * EBLK, EBLK)], dbuf)

            def ebody(i, _):
                for k in range(4):
                    off = i * 64 + k * 16
                    s = sbuf[pl.ds(off, 16)]
                    dvec = dbuf[pl.ds(off, 16)]
                    rel = dvec - base
                    m = (rel >= 0) & (rel < CH)
                    idx = jnp.where(m, rel * N_ATOM + s, 0)
                    plsc.addupdate_scatter(buf, [idx], ones, mask=m)
                return _
            lax.fori_loop(0, EBLK // 64, ebody, 0)

        pltpu.sync_copy(buf, a_out.at[pl.ds(base * N_ATOM, CH * N_ATOM)])

    mbase = wid * MCH
    pltpu.sync_copy(msrc_hbm, msbuf)
    pltpu.sync_copy(mdst_hbm, mdbuf)

    def mzbody(i, _):
        for k in range(8):
            mbuf[pl.ds(i * 128 + k * 16, 16)] = zeros
        return _
    lax.fori_loop(0, MCH * N_MOTIF // 128, mzbody, 0)

    def mebody(i, _):
        for k in range(4):
            off = i * 64 + k * 16
            s = msbuf[pl.ds(off, 16)]
            dvec = mdbuf[pl.ds(off, 16)]
            rel = dvec - mbase
            m = (rel >= 0) & (rel < MCH)
            idx = jnp.where(m, rel * N_MOTIF + s, 0)
            plsc.addupdate_scatter(mbuf, [idx], ones, mask=m)
        return _
    lax.fori_loop(0, E_MOTIF // 64, mebody, 0)

    pltpu.sync_copy(mbuf, am_out.at[pl.ds(mbase * N_MOTIF, MCH * N_MOTIF)])


def _dot(a, b, ca=1, cb=0):
    return lax.dot_general(a, b, (((ca,), (cb,)), ((), ())), precision=_PREC)


def _full(shape):
    return pl.BlockSpec(shape, lambda *_: (0,) * len(shape))


def _dinv_body(a_ref, dinv_ref):
    deg = jnp.sum(a_ref[...], axis=1, keepdims=True) + 1.0
    dinv_ref[...] = lax.rsqrt(deg)


def _dinv(A):
    return pl.pallas_call(
        _dinv_body,
        grid=(NRB,),
        in_specs=[pl.BlockSpec((RB, N_ATOM), lambda i: (i, 0))],
        out_specs=pl.BlockSpec((RB, 1), lambda i: (i, 0)),
        out_shape=jax.ShapeDtypeStruct((N_ATOM, 1), jnp.float32),
    )(A)


def _embed_body(x_ref, w_ref, b_ref, o_ref):
    o_ref[...] = _dot(x_ref[...], w_ref[...]) + b_ref[...]


def _embed(x, w, b, rb):
    n, fin = x.shape
    fout = w.shape[1]
    return pl.pallas_call(
        _embed_body,
        grid=(n // rb,),
        in_specs=[pl.BlockSpec((rb, fin), lambda i: (i, 0)),
                  _full((fin, fout)), _full((1, fout))],
        out_specs=pl.BlockSpec((rb, fout), lambda i: (i, 0)),
        out_shape=jax.ShapeDtypeStruct((n, fout), jnp.float32),
    )(x, w, b.reshape(1, fout))


def _gcn_p1_body(x_ref, w_ref, dinv_ref, o_ref):
    o_ref[...] = dinv_ref[...] * _dot(x_ref[...], w_ref[...])


def _gcn_layer(A, x, dinv, w, b):
    hd = pl.pallas_call(
        _gcn_p1_body,
        grid=(NRB,),
        in_specs=[pl.BlockSpec((RB, HIDDEN), lambda i: (i, 0)),
                  _full((HIDDEN, HIDDEN)),
                  pl.BlockSpec((RB, 1), lambda i: (i, 0))],
        out_specs=pl.BlockSpec((RB, HIDDEN), lambda i: (i, 0)),
        out_shape=jax.ShapeDtypeStruct((N_ATOM, HIDDEN), jnp.float32),
    )(x, w, dinv)

    def p2(a_ref, hdf_ref, hdb_ref, dinv_ref, b_ref, o_ref):
        t = _dot(a_ref[...], hdf_ref[...]) + hdb_ref[...]
        o_ref[...] = jnp.maximum(dinv_ref[...] * t + b_ref[...], 0.0)

    return pl.pallas_call(
        p2,
        grid=(NRB,),
        in_specs=[pl.BlockSpec((RB, N_ATOM), lambda i: (i, 0)),
                  _full((N_ATOM, HIDDEN)),
                  pl.BlockSpec((RB, HIDDEN), lambda i: (i, 0)),
                  pl.BlockSpec((RB, 1), lambda i: (i, 0)),
                  _full((1, HIDDEN))],
        out_specs=pl.BlockSpec((RB, HIDDEN), lambda i: (i, 0)),
        out_shape=jax.ShapeDtypeStruct((N_ATOM, HIDDEN), jnp.float32),
    )(A, hd, hd, dinv, b.reshape(1, HIDDEN))


def _motif_body(am_ref, mf_ref, mw_ref, mb_ref,
                w1_ref, b1_ref, w2_ref, b2_ref, o_ref):
    m = _dot(mf_ref[...], mw_ref[...]) + mb_ref[...]
    Am = am_ref[...]
    for i in range(3):
        h = m + _dot(Am, m)
        h1 = jnp.maximum(_dot(h, w1_ref[i]) + b1_ref[i][None, :], 0.0)
        m = jnp.maximum(_dot(h1, w2_ref[i]) + b2_ref[i][None, :], 0.0)
    o_ref[...] = m


def _motif_stack(Am, motif_f, mw, mb, w1, b1, w2, b2):
    return pl.pallas_call(
        _motif_body,
        out_shape=jax.ShapeDtypeStruct((N_MOTIF, HIDDEN), jnp.float32),
    )(Am, motif_f, mw, mb.reshape(1, HIDDEN), w1, b1, w2, b2)


def _qkv_body(x_ref, w_ref, b_ref, o_ref):
    o_ref[0] = _dot(x_ref[...], w_ref[...], 1, 1) + b_ref[0]


def _qkv(x, wqkv, bqkv):
    L = x.shape[0]
    return pl.pallas_call(
        _qkv_body,
        grid=(3 * HEADS,),
        in_specs=[_full((L, HIDDEN)),
                  pl.BlockSpec((HD, HIDDEN), lambda h: (h, 0)),
                  pl.BlockSpec((1, 1, HD), lambda h: (h, 0, 0))],
        out_specs=pl.BlockSpec((1, L, HD), lambda h: (h, 0, 0)),
        out_shape=jax.ShapeDtypeStruct((3 * HEADS, L, HD), jnp.float32),
    )(x, wqkv, bqkv.reshape(3 * HEADS, 1, HD))


def _attn_pool_body(q_ref, k_ref, v_ref, o_ref):
    qb = pl.program_id(1)
    L = k_ref.shape[1]
    q = q_ref[0]
    k = k_ref[0]
    v = v_ref[0]
    s = lax.dot_general(q, k, (((1,), (1,)), ((), ())),
                        precision=_PREC) * (1.0 / float(HD) ** 0.5)
    s = s - jnp.max(s, axis=1, keepdims=True)
    e = jnp.exp(s)
    p = e / jnp.sum(e, axis=1, keepdims=True)
    o = _dot(p, v)
    colsum = jnp.sum(o, axis=0, keepdims=True) * (1.0 / float(L))

    @pl.when(qb == 0)
    def _():
        o_ref[...] = jnp.zeros_like(o_ref)

    o_ref[0] += colsum


def _attn_pool(qkv, L, qb):
    return pl.pallas_call(
        _attn_pool_body,
        grid=(HEADS, L // qb),
        in_specs=[
            pl.BlockSpec((1, qb, HD), lambda h, j: (h, j, 0)),
            pl.BlockSpec((1, L, HD), lambda h, j: (HEADS + h, 0, 0)),
            pl.BlockSpec((1, L, HD), lambda h, j: (2 * HEADS + h, 0, 0)),
        ],
        out_specs=pl.BlockSpec((1, 1, HD), lambda h, j: (h, 0, 0)),
        out_shape=jax.ShapeDtypeStruct((HEADS, 1, HD), jnp.float32),
    )(qkv, qkv, qkv)


def _final_body(am_ref, mm_ref, wo_ref, bo_ref,
                w1_ref, b1_ref, w2_ref, b2_ref, o_ref):
    ag = _dot(am_ref[...], wo_ref[...], 1, 1) + bo_ref[...]
    mg = _dot(mm_ref[...], wo_ref[...], 1, 1) + bo_ref[...]
    c = jnp.concatenate([ag, mg], axis=1)
    h = jnp.maximum(_dot(c, w1_ref[...]) + b1_ref[...], 0.0)
    o_ref[...] = _dot(h, w2_ref[...]) + b2_ref[...]


def _final(atom_mean, motif_mean, wo, bo, w1, b1, w2, b2):
    return pl.pallas_call(
        _final_body,
        out_shape=jax.ShapeDtypeStruct((1, 128), jnp.float32),
    )(atom_mean, motif_mean, wo, bo.reshape(1, HIDDEN),
      w1, b1.reshape(1, HIDDEN), w2, b2.reshape(1, 128))


def _build_adjacency(src, dst, msrc, mdst):
    a_flat, am_flat = _sc_adjacency_kernel()(src, dst, msrc, mdst)
    return (a_flat.reshape(N_ATOM, N_ATOM),
            am_flat.reshape(N_MOTIF, N_MOTIF))


def kernel(atom_features, bond_features, motif_features, params,
           edge_index, motif_edge_index):
    del bond_features
    p = params
    A, Am = _build_adjacency(edge_index[0], edge_index[1],
                             motif_edge_index[0], motif_edge_index[1])

    dinv = _dinv(A)
    x = _embed(atom_features, p['atom_W'], p['atom_b'], RB)
    for i in range(3):
        x = _gcn_layer(A, x, dinv, p['gcn_W'][i], p['gcn_b'][i])

    m = _motif_stack(Am, motif_features, p['motif_W'], p['motif_b'],
                     p['gin_W1'], p['gin_b1'], p['gin_W2'], p['gin_b2'])

    qkv_a = _qkv(x, p['attn_Wqkv'], p['attn_bqkv'])
    qkv_m = _qkv(m, p['attn_Wqkv'], p['attn_bqkv'])
    am = _attn_pool(qkv_a, N_ATOM, RB).reshape(1, HIDDEN)
    mm = _attn_pool(qkv_m, N_MOTIF, RB).reshape(1, HIDDEN)

    latent = _final(am, mm, p['attn_Wo'], p['attn_bo'],
                    p['proj_W1'], p['proj_b1'], p['proj_W2'], p['proj_b2'])
    return latent.reshape(128)


# speedup vs baseline: 10.4840x; 1.1763x over previous
import functools

import jax
import jax.numpy as jnp
from jax import lax
from jax.experimental import pallas as pl
from jax.experimental.pallas import tpu as pltpu
from jax.experimental.pallas import tpu_sc as plsc

N_ATOM = 2048
E_ATOM = 65536
N_MOTIF = 512
E_MOTIF = 2048
HIDDEN = 256
HEADS = 4
HD = HIDDEN // HEADS

NW = 32
CH = 32
EBLK = 4096
MCH = N_MOTIF // NW

RB = 256
NRB = N_ATOM // RB

_PREC = jax.lax.Precision.DEFAULT


@functools.cache
def _sc_adjacency_kernel():
    mesh = plsc.VectorSubcoreMesh(core_axis_name="c", subcore_axis_name="s")
    return pl.kernel(
        _sc_adjacency_body,
        mesh=mesh,
        compiler_params=pltpu.CompilerParams(needs_layout_passes=False),
        out_type=[
            jax.ShapeDtypeStruct((N_ATOM * N_ATOM,), jnp.float32),
            jax.ShapeDtypeStruct((N_MOTIF * N_MOTIF,), jnp.float32),
        ],
        scratch_types=[
            pltpu.VMEM((CH * N_ATOM,), jnp.float32),
            pltpu.VMEM((EBLK,), jnp.int32),
            pltpu.VMEM((MCH * N_MOTIF,), jnp.float32),
            pltpu.VMEM((E_MOTIF,), jnp.int32),
        ],
    )


def _sc_adjacency_body(flat_hbm, mflat_hbm, a_out, am_out,
                       buf, fbuf, mbuf, mfbuf):
    wid = lax.axis_index("s") * 2 + lax.axis_index("c")
    ones = jnp.ones((16,), jnp.float32)
    zeros = jnp.zeros((16,), jnp.float32)
    cwords = CH * N_ATOM

    for chunk_i in range(N_ATOM // CH // NW):
        chunk = wid * (N_ATOM // CH // NW) + chunk_i
        fbase = chunk * cwords

        def zbody(i, _):
            for k in range(8):
                buf[pl.ds(i * 128 + k * 16, 16)] = zeros
            return _
        lax.fori_loop(0, cwords // 128, zbody, 0)

        for blk in range(E_ATOM // EBLK):
            pltpu.sync_copy(flat_hbm.at[pl.ds(blk * EBLK, EBLK)], fbuf)

            def ebody(i, _):
                for k in range(4):
                    rel = fbuf[pl.ds(i * 64 + k * 16, 16)] - fbase
                    m = rel.astype(jnp.uint32) < cwords
                    plsc.addupdate_scatter(buf, [rel], ones, mask=m)
                return _
            lax.fori_loop(0, EBLK // 64, ebody, 0)

        pltpu.sync_copy(buf, a_out.at[pl.ds(fbase, cwords)])

    mwords = MCH * N_MOTIF
    mfbase = wid * mwords
    pltpu.sync_copy(mflat_hbm, mfbuf)

    def mzbody(i, _):
        for k in range(8):
            mbuf[pl.ds(i * 128 + k * 16, 16)] = zeros
        return _
    lax.fori_loop(0, mwords // 128, mzbody, 0)

    def mebody(i, _):
        for k in range(4):
            rel = mfbuf[pl.ds(i * 64 + k * 16, 16)] - mfbase
            m = rel.astype(jnp.uint32) < mwords
            plsc.addupdate_scatter(mbuf, [rel], ones, mask=m)
        return _
    lax.fori_loop(0, E_MOTIF // 64, mebody, 0)

    pltpu.sync_copy(mbuf, am_out.at[pl.ds(mfbase, mwords)])


def _dot(a, b, ca=1, cb=0):
    return lax.dot_general(a, b, (((ca,), (cb,)), ((), ())), precision=_PREC)


def _full(shape):
    return pl.BlockSpec(shape, lambda *_: (0,) * len(shape))


def _flat_body(e_ref, me_ref, f_ref, mf_ref):
    f_ref[...] = e_ref[1] * N_ATOM + e_ref[0]
    mf_ref[...] = me_ref[1] * N_MOTIF + me_ref[0]


def _flat_ids(edge_index, motif_edge_index):
    f, mf = pl.pallas_call(
        _flat_body,
        out_shape=[
            jax.ShapeDtypeStruct((E_ATOM // 128, 128), jnp.int32),
            jax.ShapeDtypeStruct((E_MOTIF // 128, 128), jnp.int32),
        ],
    )(edge_index.reshape(2, E_ATOM // 128, 128),
      motif_edge_index.reshape(2, E_MOTIF // 128, 128))
    return f.reshape(E_ATOM), mf.reshape(E_MOTIF)


def _gcn_pre_body(a_ref, xf_ref, aw_ref, ab_ref, w0_ref, dinv_ref, hd_ref):
    deg = jnp.sum(a_ref[...], axis=1, keepdims=True) + 1.0
    dinv = lax.rsqrt(deg)
    dinv_ref[...] = dinv
    x0 = _dot(xf_ref[...], aw_ref[...]) + ab_ref[...]
    hd_ref[...] = dinv * _dot(x0, w0_ref[...])


def _gcn_pre(A, atom_f, aw, ab, w0):
    return pl.pallas_call(
        _gcn_pre_body,
        grid=(NRB,),
        in_specs=[pl.BlockSpec((RB, N_ATOM), lambda i: (i, 0)),
                  pl.BlockSpec((RB, 128), lambda i: (i, 0)),
                  _full((128, HIDDEN)), _full((1, HIDDEN)),
                  _full((HIDDEN, HIDDEN))],
        out_specs=[pl.BlockSpec((RB, 1), lambda i: (i, 0)),
                   pl.BlockSpec((RB, HIDDEN), lambda i: (i, 0))],
        out_shape=[jax.ShapeDtypeStruct((N_ATOM, 1), jnp.float32),
                   jax.ShapeDtypeStruct((N_ATOM, HIDDEN), jnp.float32)],
    )(A, atom_f, aw, ab.reshape(1, HIDDEN), w0)


def _gcn_fused(A, hd, dinv, b, w_next):
    specs = [pl.BlockSpec((RB, N_ATOM), lambda i: (i, 0)),
             _full((N_ATOM, HIDDEN)),
             pl.BlockSpec((RB, HIDDEN), lambda i: (i, 0)),
             pl.BlockSpec((RB, 1), lambda i: (i, 0)),
             _full((1, HIDDEN))]
    args = [A, hd, hd, dinv, b.reshape(1, HIDDEN)]
    if w_next is None:
        body = _gcn_fused_last
    else:
        specs.append(_full((HIDDEN, HIDDEN)))
        args.append(w_next)
        body = _gcn_fused_mid
    return pl.pallas_call(
        body,
        grid=(NRB,),
        in_specs=specs,
        out_specs=pl.BlockSpec((RB, HIDDEN), lambda i: (i, 0)),
        out_shape=jax.ShapeDtypeStruct((N_ATOM, HIDDEN), jnp.float32),
    )(*args)


def _gcn_fused_mid(a_ref, hdf_ref, hdb_ref, dinv_ref, b_ref, wn_ref, o_ref):
    t = _dot(a_ref[...], hdf_ref[...]) + hdb_ref[...]
    x = jnp.maximum(dinv_ref[...] * t + b_ref[...], 0.0)
    o_ref[...] = dinv_ref[...] * _dot(x, wn_ref[...])


def _gcn_fused_last(a_ref, hdf_ref, hdb_ref, dinv_ref, b_ref, o_ref):
    t = _dot(a_ref[...], hdf_ref[...]) + hdb_ref[...]
    o_ref[...] = jnp.maximum(dinv_ref[...] * t + b_ref[...], 0.0)


def _motif_body(am_ref, mf_ref, mw_ref, mb_ref,
                w1_ref, b1_ref, w2_ref, b2_ref, o_ref):
    m = _dot(mf_ref[...], mw_ref[...]) + mb_ref[...]
    Am = am_ref[...]
    for i in range(3):
        h = m + _dot(Am, m)
        h1 = jnp.maximum(_dot(h, w1_ref[i]) + b1_ref[i][None, :], 0.0)
        m = jnp.maximum(_dot(h1, w2_ref[i]) + b2_ref[i][None, :], 0.0)
    o_ref[...] = m


def _motif_stack(Am, motif_f, mw, mb, w1, b1, w2, b2):
    return pl.pallas_call(
        _motif_body,
        out_shape=jax.ShapeDtypeStruct((N_MOTIF, HIDDEN), jnp.float32),
    )(Am, motif_f, mw, mb.reshape(1, HIDDEN), w1, b1, w2, b2)


def _qkv_body(x_ref, w_ref, b_ref, o_ref):
    o_ref[0] = _dot(x_ref[...], w_ref[...], 1, 1) + b_ref[0]


def _qkv(x, wqkv, bqkv):
    L = x.shape[0]
    return pl.pallas_call(
        _qkv_body,
        grid=(3 * HEADS,),
        in_specs=[_full((L, HIDDEN)),
                  pl.BlockSpec((HD, HIDDEN), lambda h: (h, 0)),
                  pl.BlockSpec((1, 1, HD), lambda h: (h, 0, 0))],
        out_specs=pl.BlockSpec((1, L, HD), lambda h: (h, 0, 0)),
        out_shape=jax.ShapeDtypeStruct((3 * HEADS, L, HD), jnp.float32),
    )(x, wqkv, bqkv.reshape(3 * HEADS, 1, HD))


def _attn_pool_body(q_ref, k_ref, v_ref, o_ref):
    qb = pl.program_id(1)
    L = k_ref.shape[1]
    q = q_ref[0]
    k = k_ref[0]
    v = v_ref[0]
    s = lax.dot_general(q, k, (((1,), (1,)), ((), ())),
                        precision=_PREC) * (1.0 / float(HD) ** 0.5)
    s = s - jnp.max(s, axis=1, keepdims=True)
    e = jnp.exp(s)
    p = e / jnp.sum(e, axis=1, keepdims=True)
    o = _dot(p, v)
    colsum = jnp.sum(o, axis=0, keepdims=True) * (1.0 / float(L))

    @pl.when(qb == 0)
    def _():
        o_ref[...] = jnp.zeros_like(o_ref)

    o_ref[0] += colsum


def _attn_pool(qkv, L, qb):
    return pl.pallas_call(
        _attn_pool_body,
        grid=(HEADS, L // qb),
        in_specs=[
            pl.BlockSpec((1, qb, HD), lambda h, j: (h, j, 0)),
            pl.BlockSpec((1, L, HD), lambda h, j: (HEADS + h, 0, 0)),
            pl.BlockSpec((1, L, HD), lambda h, j: (2 * HEADS + h, 0, 0)),
        ],
        out_specs=pl.BlockSpec((1, 1, HD), lambda h, j: (h, 0, 0)),
        out_shape=jax.ShapeDtypeStruct((HEADS, 1, HD), jnp.float32),
    )(qkv, qkv, qkv)


def _final_body(am_ref, mm_ref, wo_ref, bo_ref,
                w1_ref, b1_ref, w2_ref, b2_ref, o_ref):
    ag = _dot(am_ref[...], wo_ref[...], 1, 1) + bo_ref[...]
    mg = _dot(mm_ref[...], wo_ref[...], 1, 1) + bo_ref[...]
    c = jnp.concatenate([ag, mg], axis=1)
    h = jnp.maximum(_dot(c, w1_ref[...]) + b1_ref[...], 0.0)
    o_ref[...] = _dot(h, w2_ref[...]) + b2_ref[...]


def _final(atom_mean, motif_mean, wo, bo, w1, b1, w2, b2):
    return pl.pallas_call(
        _final_body,
        out_shape=jax.ShapeDtypeStruct((1, 128), jnp.float32),
    )(atom_mean, motif_mean, wo, bo.reshape(1, HIDDEN),
      w1, b1.reshape(1, HIDDEN), w2, b2.reshape(1, 128))


def _build_adjacency(edge_index, motif_edge_index):
    flat, mflat = _flat_ids(edge_index, motif_edge_index)
    a_flat, am_flat = _sc_adjacency_kernel()(flat, mflat)
    return (a_flat.reshape(N_ATOM, N_ATOM),
            am_flat.reshape(N_MOTIF, N_MOTIF))


def kernel(atom_features, bond_features, motif_features, params,
           edge_index, motif_edge_index):
    del bond_features
    p = params
    A, Am = _build_adjacency(edge_index, motif_edge_index)

    dinv, hd = _gcn_pre(A, atom_features, p['atom_W'], p['atom_b'],
                        p['gcn_W'][0])
    hd = _gcn_fused(A, hd, dinv, p['gcn_b'][0], p['gcn_W'][1])
    hd = _gcn_fused(A, hd, dinv, p['gcn_b'][1], p['gcn_W'][2])
    x = _gcn_fused(A, hd, dinv, p['gcn_b'][2], None)

    m = _motif_stack(Am, motif_features, p['motif_W'], p['motif_b'],
                     p['gin_W1'], p['gin_b1'], p['gin_W2'], p['gin_b2'])

    qkv_a = _qkv(x, p['attn_Wqkv'], p['attn_bqkv'])
    qkv_m = _qkv(m, p['attn_Wqkv'], p['attn_bqkv'])
    am = _attn_pool(qkv_a, N_ATOM, RB).reshape(1, HIDDEN)
    mm = _attn_pool(qkv_m, N_MOTIF, RB).reshape(1, HIDDEN)

    latent = _final(am, mm, p['attn_Wo'], p['attn_bo'],
                    p['proj_W1'], p['proj_b1'], p['proj_W2'], p['proj_b2'])
    return latent.reshape(128)


# speedup vs baseline: 14.3070x; 1.3646x over previous
import functools

import jax
import jax.numpy as jnp
from jax import lax
from jax.experimental import pallas as pl
from jax.experimental.pallas import tpu as pltpu
from jax.experimental.pallas import tpu_sc as plsc

N_ATOM = 2048
E_ATOM = 65536
N_MOTIF = 512
E_MOTIF = 2048
HIDDEN = 256
HEADS = 4
HD = HIDDEN // HEADS

NW = 32
CH = 64
PCOL = N_ATOM // 2
CWORDS = CH * PCOL
EBLK = 4096
MCH = N_MOTIF // NW

RB = 256
NRB = N_ATOM // RB

_PREC = jax.lax.Precision.DEFAULT


@functools.cache
def _sc_adjacency_kernel():
    mesh = plsc.VectorSubcoreMesh(core_axis_name="c", subcore_axis_name="s")
    return pl.kernel(
        _sc_adjacency_body,
        mesh=mesh,
        compiler_params=pltpu.CompilerParams(needs_layout_passes=False),
        out_type=[
            jax.ShapeDtypeStruct((N_ATOM * PCOL,), jnp.int32),
            jax.ShapeDtypeStruct((N_MOTIF * N_MOTIF,), jnp.float32),
        ],
        scratch_types=[
            pltpu.VMEM((CWORDS,), jnp.int32),
            pltpu.VMEM((EBLK,), jnp.int32),
            pltpu.VMEM((EBLK,), jnp.int32),
            pltpu.VMEM((EBLK,), jnp.int32),
            pltpu.VMEM((EBLK,), jnp.int32),
            pltpu.VMEM((MCH * N_MOTIF,), jnp.float32),
            pltpu.VMEM((E_MOTIF,), jnp.int32),
            pltpu.SemaphoreType.DMA,
            pltpu.SemaphoreType.DMA,
        ],
    )


def _sc_adjacency_body(flat_hbm, val_hbm, mflat_hbm, a_out, am_out,
                       buf, fbuf0, fbuf1, vbuf0, vbuf1, mbuf, mfbuf,
                       sem0, sem1):
    wid = lax.axis_index("s") * 2 + lax.axis_index("c")
    ones = jnp.ones((16,), jnp.float32)
    zeros = jnp.zeros((16,), jnp.float32)
    izeros = jnp.zeros((16,), jnp.int32)
    fbase = wid * CWORDS
    fbufs, vbufs, sems = (fbuf0, fbuf1), (vbuf0, vbuf1), (sem0, sem1)
    nblk = E_ATOM // EBLK

    def _start(blk):
        slot = blk % 2
        return (
            pltpu.async_copy(flat_hbm.at[pl.ds(blk * EBLK, EBLK)],
                             fbufs[slot], sems[slot]),
            pltpu.async_copy(val_hbm.at[pl.ds(blk * EBLK, EBLK)],
                             vbufs[slot], sems[slot]),
        )

    pending = _start(0)

    def zbody(i, _):
        for k in range(8):
            buf[pl.ds(i * 128 + k * 16, 16)] = izeros
        return _
    lax.fori_loop(0, CWORDS // 128, zbody, 0)

    for blk in range(nblk):
        slot = blk % 2
        for h in pending:
            h.wait()
        if blk + 1 < nblk:
            pending = _start(blk + 1)
        fbuf, vbuf = fbufs[slot], vbufs[slot]

        def ebody(i, _):
            for k in range(4):
                off = i * 64 + k * 16
                rel = fbuf[pl.ds(off, 16)] - fbase
                m = rel.astype(jnp.uint32) < CWORDS
                plsc.addupdate_scatter(buf, [rel], vbuf[pl.ds(off, 16)],
                                       mask=m)
            return _
        lax.fori_loop(0, EBLK // 64, ebody, 0)

    pltpu.sync_copy(buf, a_out.at[pl.ds(fbase, CWORDS)])

    mwords = MCH * N_MOTIF
    mfbase = wid * mwords
    pltpu.sync_copy(mflat_hbm, mfbuf)

    def mzbody(i, _):
        for k in range(8):
            mbuf[pl.ds(i * 128 + k * 16, 16)] = zeros
        return _
    lax.fori_loop(0, mwords // 128, mzbody, 0)

    def mebody(i, _):
        for k in range(4):
            rel = mfbuf[pl.ds(i * 64 + k * 16, 16)] - mfbase
            m = rel.astype(jnp.uint32) < mwords
            plsc.addupdate_scatter(mbuf, [rel], ones, mask=m)
        return _
    lax.fori_loop(0, E_MOTIF // 64, mebody, 0)

    pltpu.sync_copy(mbuf, am_out.at[pl.ds(mfbase, mwords)])


def _dot(a, b, ca=1, cb=0):
    return lax.dot_general(a, b, (((ca,), (cb,)), ((), ())), precision=_PREC)


def _full(shape):
    return pl.BlockSpec(shape, lambda *_: (0,) * len(shape))


def _flat_body(e_ref, me_ref, f_ref, v_ref, mf_ref):
    src, dst = e_ref[0], e_ref[1]
    f_ref[...] = dst * PCOL + (src & (PCOL - 1))
    v_ref[...] = 1 << ((src >> 10) << 4)
    mf_ref[...] = me_ref[1] * N_MOTIF + me_ref[0]


def _flat_ids(edge_index, motif_edge_index):
    f, v, mf = pl.pallas_call(
        _flat_body,
        out_shape=[
            jax.ShapeDtypeStruct((E_ATOM // 128, 128), jnp.int32),
            jax.ShapeDtypeStruct((E_ATOM // 128, 128), jnp.int32),
            jax.ShapeDtypeStruct((E_MOTIF // 128, 128), jnp.int32),
        ],
    )(edge_index.reshape(2, E_ATOM // 128, 128),
      motif_edge_index.reshape(2, E_MOTIF // 128, 128))
    return f.reshape(E_ATOM), v.reshape(E_ATOM), mf.reshape(E_MOTIF)


def _unpack(a_packed):
    low = (a_packed & 0xFFFF).astype(jnp.float32)
    high = (a_packed >> 16).astype(jnp.float32)
    return low, high


def _gcn_pre_body(a_ref, xf_ref, aw_ref, ab_ref, w0_ref, dinv_ref, hd_ref):
    low, high = _unpack(a_ref[...])
    deg = (jnp.sum(low, axis=1, keepdims=True)
           + jnp.sum(high, axis=1, keepdims=True) + 1.0)
    dinv = lax.rsqrt(deg)
    dinv_ref[...] = dinv
    x0 = _dot(xf_ref[...], aw_ref[...]) + ab_ref[...]
    hd_ref[...] = dinv * _dot(x0, w0_ref[...])


def _gcn_pre(A, atom_f, aw, ab, w0):
    return pl.pallas_call(
        _gcn_pre_body,
        grid=(NRB,),
        in_specs=[pl.BlockSpec((RB, PCOL), lambda i: (i, 0)),
                  pl.BlockSpec((RB, 128), lambda i: (i, 0)),
                  _full((128, HIDDEN)), _full((1, HIDDEN)),
                  _full((HIDDEN, HIDDEN))],
        out_specs=[pl.BlockSpec((RB, 1), lambda i: (i, 0)),
                   pl.BlockSpec((RB, HIDDEN), lambda i: (i, 0))],
        out_shape=[jax.ShapeDtypeStruct((N_ATOM, 1), jnp.float32),
                   jax.ShapeDtypeStruct((N_ATOM, HIDDEN), jnp.float32)],
    )(A, atom_f, aw, ab.reshape(1, HIDDEN), w0)


def _gcn_fused(A, hd, dinv, b, w_next):
    specs = [pl.BlockSpec((RB, PCOL), lambda i: (i, 0)),
             pl.BlockSpec((PCOL, HIDDEN), lambda i: (0, 0)),
             pl.BlockSpec((PCOL, HIDDEN), lambda i: (1, 0)),
             pl.BlockSpec((RB, HIDDEN), lambda i: (i, 0)),
             pl.BlockSpec((RB, 1), lambda i: (i, 0)),
             _full((1, HIDDEN))]
    args = [A, hd, hd, hd, dinv, b.reshape(1, HIDDEN)]
    if w_next is None:
        body = _gcn_fused_last
    else:
        specs.append(_full((HIDDEN, HIDDEN)))
        args.append(w_next)
        body = _gcn_fused_mid
    return pl.pallas_call(
        body,
        grid=(NRB,),
        in_specs=specs,
        out_specs=pl.BlockSpec((RB, HIDDEN), lambda i: (i, 0)),
        out_shape=jax.ShapeDtypeStruct((N_ATOM, HIDDEN), jnp.float32),
    )(*args)


def _gcn_agg(a_ref, hdt_ref, hdu_ref, hdb_ref):
    low, high = _unpack(a_ref[...])
    return (_dot(low, hdt_ref[...]) + _dot(high, hdu_ref[...])
            + hdb_ref[...])


def _gcn_fused_mid(a_ref, hdt_ref, hdu_ref, hdb_ref, dinv_ref, b_ref,
                   wn_ref, o_ref):
    t = _gcn_agg(a_ref, hdt_ref, hdu_ref, hdb_ref)
    x = jnp.maximum(dinv_ref[...] * t + b_ref[...], 0.0)
    o_ref[...] = dinv_ref[...] * _dot(x, wn_ref[...])


def _gcn_fused_last(a_ref, hdt_ref, hdu_ref, hdb_ref, dinv_ref, b_ref,
                    o_ref):
    t = _gcn_agg(a_ref, hdt_ref, hdu_ref, hdb_ref)
    o_ref[...] = jnp.maximum(dinv_ref[...] * t + b_ref[...], 0.0)


def _motif_body(am_ref, mf_ref, mw_ref, mb_ref,
                w1_ref, b1_ref, w2_ref, b2_ref, o_ref):
    m = _dot(mf_ref[...], mw_ref[...]) + mb_ref[...]
    Am = am_ref[...]
    for i in range(3):
        h = m + _dot(Am, m)
        h1 = jnp.maximum(_dot(h, w1_ref[i]) + b1_ref[i][None, :], 0.0)
        m = jnp.maximum(_dot(h1, w2_ref[i]) + b2_ref[i][None, :], 0.0)
    o_ref[...] = m


def _motif_stack(Am, motif_f, mw, mb, w1, b1, w2, b2):
    return pl.pallas_call(
        _motif_body,
        out_shape=jax.ShapeDtypeStruct((N_MOTIF, HIDDEN), jnp.float32),
    )(Am, motif_f, mw, mb.reshape(1, HIDDEN), w1, b1, w2, b2)


def _qkv_body(x_ref, w_ref, b_ref, o_ref):
    o_ref[0] = _dot(x_ref[...], w_ref[...], 1, 1) + b_ref[0]


def _qkv(x, wqkv, bqkv):
    L = x.shape[0]
    return pl.pallas_call(
        _qkv_body,
        grid=(3 * HEADS,),
        in_specs=[_full((L, HIDDEN)),
                  pl.BlockSpec((HD, HIDDEN), lambda h: (h, 0)),
                  pl.BlockSpec((1, 1, HD), lambda h: (h, 0, 0))],
        out_specs=pl.BlockSpec((1, L, HD), lambda h: (h, 0, 0)),
        out_shape=jax.ShapeDtypeStruct((3 * HEADS, L, HD), jnp.float32),
    )(x, wqkv, bqkv.reshape(3 * HEADS, 1, HD))


def _attn_pool_body(q_ref, k_ref, v_ref, o_ref):
    qb = pl.program_id(1)
    L = k_ref.shape[1]
    q = q_ref[0]
    k = k_ref[0]
    v = v_ref[0]
    s = lax.dot_general(q, k, (((1,), (1,)), ((), ())),
                        precision=_PREC) * (1.0 / float(HD) ** 0.5)
    s = s - jnp.max(s, axis=1, keepdims=True)
    e = jnp.exp(s)
    p = e / jnp.sum(e, axis=1, keepdims=True)
    o = _dot(p, v)
    colsum = jnp.sum(o, axis=0, keepdims=True) * (1.0 / float(L))

    @pl.when(qb == 0)
    def _():
        o_ref[...] = jnp.zeros_like(o_ref)

    o_ref[0] += colsum


def _attn_pool(qkv, L, qb):
    return pl.pallas_call(
        _attn_pool_body,
        grid=(HEADS, L // qb),
        in_specs=[
            pl.BlockSpec((1, qb, HD), lambda h, j: (h, j, 0)),
            pl.BlockSpec((1, L, HD), lambda h, j: (HEADS + h, 0, 0)),
            pl.BlockSpec((1, L, HD), lambda h, j: (2 * HEADS + h, 0, 0)),
        ],
        out_specs=pl.BlockSpec((1, 1, HD), lambda h, j: (h, 0, 0)),
        out_shape=jax.ShapeDtypeStruct((HEADS, 1, HD), jnp.float32),
    )(qkv, qkv, qkv)


def _final_body(am_ref, mm_ref, wo_ref, bo_ref,
                w1_ref, b1_ref, w2_ref, b2_ref, o_ref):
    ag = _dot(am_ref[...], wo_ref[...], 1, 1) + bo_ref[...]
    mg = _dot(mm_ref[...], wo_ref[...], 1, 1) + bo_ref[...]
    c = jnp.concatenate([ag, mg], axis=1)
    h = jnp.maximum(_dot(c, w1_ref[...]) + b1_ref[...], 0.0)
    o_ref[...] = _dot(h, w2_ref[...]) + b2_ref[...]


def _final(atom_mean, motif_mean, wo, bo, w1, b1, w2, b2):
    return pl.pallas_call(
        _final_body,
        out_shape=jax.ShapeDtypeStruct((1, 128), jnp.float32),
    )(atom_mean, motif_mean, wo, bo.reshape(1, HIDDEN),
      w1, b1.reshape(1, HIDDEN), w2, b2.reshape(1, 128))


def _build_adjacency(edge_index, motif_edge_index):
    flat, vals, mflat = _flat_ids(edge_index, motif_edge_index)
    a_flat, am_flat = _sc_adjacency_kernel()(flat, vals, mflat)
    return (a_flat.reshape(N_ATOM, PCOL),
            am_flat.reshape(N_MOTIF, N_MOTIF))


def kernel(atom_features, bond_features, motif_features, params,
           edge_index, motif_edge_index):
    del bond_features
    p = params
    A, Am = _build_adjacency(edge_index, motif_edge_index)

    dinv, hd = _gcn_pre(A, atom_features, p['atom_W'], p['atom_b'],
                        p['gcn_W'][0])
    hd = _gcn_fused(A, hd, dinv, p['gcn_b'][0], p['gcn_W'][1])
    hd = _gcn_fused(A, hd, dinv, p['gcn_b'][1], p['gcn_W'][2])
    x = _gcn_fused(A, hd, dinv, p['gcn_b'][2], None)

    m = _motif_stack(Am, motif_features, p['motif_W'], p['motif_b'],
                     p['gin_W1'], p['gin_b1'], p['gin_W2'], p['gin_b2'])

    qkv_a = _qkv(x, p['attn_Wqkv'], p['attn_bqkv'])
    qkv_m = _qkv(m, p['attn_Wqkv'], p['attn_bqkv'])
    am = _attn_pool(qkv_a, N_ATOM, RB).reshape(1, HIDDEN)
    mm = _attn_pool(qkv_m, N_MOTIF, RB).reshape(1, HIDDEN)

    latent = _final(am, mm, p['attn_Wo'], p['attn_bo'],
                    p['proj_W1'], p['proj_b1'], p['proj_W2'], p['proj_b2'])
    return latent.reshape(128)


# speedup vs baseline: 16.0640x; 1.1228x over previous
import functools

import jax
import jax.numpy as jnp
from jax import lax
from jax.experimental import pallas as pl
from jax.experimental.pallas import tpu as pltpu
from jax.experimental.pallas import tpu_sc as plsc

N_ATOM = 2048
E_ATOM = 65536
N_MOTIF = 512
E_MOTIF = 2048
HIDDEN = 256
HEADS = 4
HD = HIDDEN // HEADS

NW = 32
CH = 64
PCOL = N_ATOM // 2
CWORDS = CH * PCOL
EBLK = 4096
MCH = N_MOTIF // NW

RB = 256
NRB = N_ATOM // RB

_PREC = jax.lax.Precision.DEFAULT


@functools.cache
def _sc_adjacency_kernel():
    mesh = plsc.VectorSubcoreMesh(core_axis_name="c", subcore_axis_name="s")
    return pl.kernel(
        _sc_adjacency_body,
        mesh=mesh,
        compiler_params=pltpu.CompilerParams(needs_layout_passes=False),
        out_type=[
            jax.ShapeDtypeStruct((N_ATOM * PCOL,), jnp.int32),
            jax.ShapeDtypeStruct((N_MOTIF * N_MOTIF,), jnp.float32),
        ],
        scratch_types=[
            pltpu.VMEM((CWORDS,), jnp.int32),
            pltpu.VMEM((EBLK,), jnp.int32),
            pltpu.VMEM((EBLK,), jnp.int32),
            pltpu.VMEM((EBLK,), jnp.int32),
            pltpu.VMEM((EBLK,), jnp.int32),
            pltpu.VMEM((MCH * N_MOTIF,), jnp.float32),
            pltpu.VMEM((E_MOTIF,), jnp.int32),
            pltpu.SemaphoreType.DMA,
            pltpu.SemaphoreType.DMA,
        ],
    )


def _sc_adjacency_body(flat_hbm, val_hbm, mflat_hbm, a_out, am_out,
                       buf, fbuf0, fbuf1, vbuf0, vbuf1, mbuf, mfbuf,
                       sem0, sem1):
    wid = lax.axis_index("s") * 2 + lax.axis_index("c")
    ones = jnp.ones((16,), jnp.float32)
    zeros = jnp.zeros((16,), jnp.float32)
    izeros = jnp.zeros((16,), jnp.int32)
    fbase = wid * CWORDS
    fbufs, vbufs, sems = (fbuf0, fbuf1), (vbuf0, vbuf1), (sem0, sem1)
    nblk = E_ATOM // EBLK

    def _start(blk):
        slot = blk % 2
        return (
            pltpu.async_copy(flat_hbm.at[pl.ds(blk * EBLK, EBLK)],
                             fbufs[slot], sems[slot]),
            pltpu.async_copy(val_hbm.at[pl.ds(blk * EBLK, EBLK)],
                             vbufs[slot], sems[slot]),
        )

    pending = _start(0)

    def zbody(i, _):
        for k in range(8):
            buf[pl.ds(i * 128 + k * 16, 16)] = izeros
        return _
    lax.fori_loop(0, CWORDS // 128, zbody, 0)

    for blk in range(nblk):
        slot = blk % 2
        for h in pending:
            h.wait()
        if blk + 1 < nblk:
            pending = _start(blk + 1)
        fbuf, vbuf = fbufs[slot], vbufs[slot]

        def ebody(i, _):
            for k in range(4):
                off = i * 64 + k * 16
                rel = fbuf[pl.ds(off, 16)] - fbase
                m = rel.astype(jnp.uint32) < CWORDS
                plsc.addupdate_scatter(buf, [rel], vbuf[pl.ds(off, 16)],
                                       mask=m)
            return _
        lax.fori_loop(0, EBLK // 64, ebody, 0)

    pltpu.sync_copy(buf, a_out.at[pl.ds(fbase, CWORDS)])

    mwords = MCH * N_MOTIF
    mfbase = wid * mwords
    pltpu.sync_copy(mflat_hbm, mfbuf)

    def mzbody(i, _):
        for k in range(8):
            mbuf[pl.ds(i * 128 + k * 16, 16)] = zeros
        return _
    lax.fori_loop(0, mwords // 128, mzbody, 0)

    def mebody(i, _):
        for k in range(4):
            rel = mfbuf[pl.ds(i * 64 + k * 16, 16)] - mfbase
            m = rel.astype(jnp.uint32) < mwords
            plsc.addupdate_scatter(mbuf, [rel], ones, mask=m)
        return _
    lax.fori_loop(0, E_MOTIF // 64, mebody, 0)

    pltpu.sync_copy(mbuf, am_out.at[pl.ds(mfbase, mwords)])


def _dot(a, b, ca=1, cb=0):
    return lax.dot_general(a, b, (((ca,), (cb,)), ((), ())), precision=_PREC)


def _full(shape):
    return pl.BlockSpec(shape, lambda *_: (0,) * len(shape))


def _flat_body(e_ref, me_ref, f_ref, v_ref, mf_ref):
    src, dst = e_ref[0], e_ref[1]
    f_ref[...] = dst * PCOL + (src & (PCOL - 1))
    v_ref[...] = 1 << ((src >> 10) << 4)
    mf_ref[...] = me_ref[1] * N_MOTIF + me_ref[0]


def _flat_ids(edge_index, motif_edge_index):
    f, v, mf = pl.pallas_call(
        _flat_body,
        out_shape=[
            jax.ShapeDtypeStruct((E_ATOM // 128, 128), jnp.int32),
            jax.ShapeDtypeStruct((E_ATOM // 128, 128), jnp.int32),
            jax.ShapeDtypeStruct((E_MOTIF // 128, 128), jnp.int32),
        ],
    )(edge_index.reshape(2, E_ATOM // 128, 128),
      motif_edge_index.reshape(2, E_MOTIF // 128, 128))
    return f.reshape(E_ATOM), v.reshape(E_ATOM), mf.reshape(E_MOTIF)


def _unpack(a_packed):
    low = (a_packed & 0xFFFF).astype(jnp.float32)
    high = (a_packed >> 16).astype(jnp.float32)
    return low, high


def _gcn_pre_body(a_ref, xf_ref, aw_ref, ab_ref, w0_ref, dinv_ref, hd_ref):
    low, high = _unpack(a_ref[...])
    deg = (jnp.sum(low, axis=1, keepdims=True)
           + jnp.sum(high, axis=1, keepdims=True) + 1.0)
    dinv = lax.rsqrt(deg)
    dinv_ref[...] = dinv
    x0 = _dot(xf_ref[...], aw_ref[...]) + ab_ref[...]
    hd_ref[...] = dinv * _dot(x0, w0_ref[...])


def _gcn_pre(A, atom_f, aw, ab, w0):
    return pl.pallas_call(
        _gcn_pre_body,
        grid=(NRB,),
        in_specs=[pl.BlockSpec((RB, PCOL), lambda i: (i, 0)),
                  pl.BlockSpec((RB, 128), lambda i: (i, 0)),
                  _full((128, HIDDEN)), _full((1, HIDDEN)),
                  _full((HIDDEN, HIDDEN))],
        out_specs=[pl.BlockSpec((RB, 1), lambda i: (i, 0)),
                   pl.BlockSpec((RB, HIDDEN), lambda i: (i, 0))],
        out_shape=[jax.ShapeDtypeStruct((N_ATOM, 1), jnp.float32),
                   jax.ShapeDtypeStruct((N_ATOM, HIDDEN), jnp.float32)],
    )(A, atom_f, aw, ab.reshape(1, HIDDEN), w0)


def _gcn_fused(A, hd, dinv, b, w_next):
    specs = [pl.BlockSpec((RB, PCOL), lambda i: (i, 0)),
             pl.BlockSpec((PCOL, HIDDEN), lambda i: (0, 0)),
             pl.BlockSpec((PCOL, HIDDEN), lambda i: (1, 0)),
             pl.BlockSpec((RB, HIDDEN), lambda i: (i, 0)),
             pl.BlockSpec((RB, 1), lambda i: (i, 0)),
             _full((1, HIDDEN))]
    args = [A, hd, hd, hd, dinv, b.reshape(1, HIDDEN)]
    if w_next is None:
        body = _gcn_fused_last
    else:
        specs.append(_full((HIDDEN, HIDDEN)))
        args.append(w_next)
        body = _gcn_fused_mid
    return pl.pallas_call(
        body,
        grid=(NRB,),
        in_specs=specs,
        out_specs=pl.BlockSpec((RB, HIDDEN), lambda i: (i, 0)),
        out_shape=jax.ShapeDtypeStruct((N_ATOM, HIDDEN), jnp.float32),
    )(*args)


def _gcn_agg(a_ref, hdt_ref, hdu_ref, hdb_ref):
    low, high = _unpack(a_ref[...])
    return (_dot(low, hdt_ref[...]) + _dot(high, hdu_ref[...])
            + hdb_ref[...])


def _gcn_fused_mid(a_ref, hdt_ref, hdu_ref, hdb_ref, dinv_ref, b_ref,
                   wn_ref, o_ref):
    t = _gcn_agg(a_ref, hdt_ref, hdu_ref, hdb_ref)
    x = jnp.maximum(dinv_ref[...] * t + b_ref[...], 0.0)
    o_ref[...] = dinv_ref[...] * _dot(x, wn_ref[...])


def _gcn_fused_last(a_ref, hdt_ref, hdu_ref, hdb_ref, dinv_ref, b_ref,
                    o_ref):
    t = _gcn_agg(a_ref, hdt_ref, hdu_ref, hdb_ref)
    o_ref[...] = jnp.maximum(dinv_ref[...] * t + b_ref[...], 0.0)


def _softmax_rows(s):
    s = s - jnp.max(s, axis=1, keepdims=True)
    e = jnp.exp(s)
    return e / jnp.sum(e, axis=1, keepdims=True)


def _motif_body(am_ref, mf_ref, mw_ref, mb_ref,
                w1_ref, b1_ref, w2_ref, b2_ref,
                wqkv_ref, bqkv_ref, o_ref):
    m = _dot(mf_ref[...], mw_ref[...]) + mb_ref[...]
    Am = am_ref[...]
    for i in range(3):
        h = m + _dot(Am, m)
        h1 = jnp.maximum(_dot(h, w1_ref[i]) + b1_ref[i][None, :], 0.0)
        m = jnp.maximum(_dot(h1, w2_ref[i]) + b2_ref[i][None, :], 0.0)
    qkv = _dot(m, wqkv_ref[...], 1, 1) + bqkv_ref[...]
    outs = []
    for h in range(HEADS):
        q = qkv[:, h * HD:(h + 1) * HD]
        k = qkv[:, HIDDEN + h * HD:HIDDEN + (h + 1) * HD]
        v = qkv[:, 2 * HIDDEN + h * HD:2 * HIDDEN + (h + 1) * HD]
        s = lax.dot_general(q, k, (((1,), (1,)), ((), ())),
                            precision=_PREC) * (1.0 / float(HD) ** 0.5)
        p = _softmax_rows(s)
        o = _dot(p, v)
        outs.append(jnp.sum(o, axis=0, keepdims=True) * (1.0 / N_MOTIF))
    o_ref[...] = jnp.concatenate(outs, axis=1)


def _motif_pooled(Am, motif_f, mw, mb, w1, b1, w2, b2, wqkv, bqkv):
    return pl.pallas_call(
        _motif_body,
        out_shape=jax.ShapeDtypeStruct((1, HIDDEN), jnp.float32),
    )(Am, motif_f, mw, mb.reshape(1, HIDDEN), w1, b1, w2, b2,
      wqkv, bqkv.reshape(1, 3 * HIDDEN))


def _attn_pool_body(xb_ref, xf_ref, wq_ref, wk_ref, wv_ref,
                    bq_ref, bk_ref, bv_ref, o_ref, k_s, v_s):
    j = pl.program_id(1)
    L = xf_ref.shape[0]

    @pl.when(j == 0)
    def _():
        k_s[...] = _dot(xf_ref[...], wk_ref[...], 1, 1) + bk_ref[0]
        v_s[...] = _dot(xf_ref[...], wv_ref[...], 1, 1) + bv_ref[0]

    q = _dot(xb_ref[...], wq_ref[...], 1, 1) + bq_ref[0]
    s = lax.dot_general(q, k_s[...], (((1,), (1,)), ((), ())),
                        precision=_PREC) * (1.0 / float(HD) ** 0.5)
    p = _softmax_rows(s)
    o = _dot(p, v_s[...])
    colsum = jnp.sum(o, axis=0, keepdims=True) * (1.0 / float(L))

    @pl.when(j == 0)
    def _():
        o_ref[...] = jnp.zeros_like(o_ref)

    o_ref[0] += colsum


def _attn_pool(x, wqkv, bqkv):
    L = x.shape[0]
    b3 = bqkv.reshape(3 * HEADS, 1, HD)
    return pl.pallas_call(
        _attn_pool_body,
        grid=(HEADS, L // RB),
        in_specs=[
            pl.BlockSpec((RB, HIDDEN), lambda h, j: (j, 0)),
            _full((L, HIDDEN)),
            pl.BlockSpec((HD, HIDDEN), lambda h, j: (h, 0)),
            pl.BlockSpec((HD, HIDDEN), lambda h, j: (HEADS + h, 0)),
            pl.BlockSpec((HD, HIDDEN), lambda h, j: (2 * HEADS + h, 0)),
            pl.BlockSpec((1, 1, HD), lambda h, j: (h, 0, 0)),
            pl.BlockSpec((1, 1, HD), lambda h, j: (HEADS + h, 0, 0)),
            pl.BlockSpec((1, 1, HD), lambda h, j: (2 * HEADS + h, 0, 0)),
        ],
        out_specs=pl.BlockSpec((1, 1, HD), lambda h, j: (h, 0, 0)),
        out_shape=jax.ShapeDtypeStruct((HEADS, 1, HD), jnp.float32),
        scratch_shapes=[pltpu.VMEM((L, HD), jnp.float32),
                        pltpu.VMEM((L, HD), jnp.float32)],
    )(x, x, wqkv, wqkv, wqkv, b3, b3, b3)


def _final_body(am_ref, mm_ref, wo_ref, bo_ref,
                w1_ref, b1_ref, w2_ref, b2_ref, o_ref):
    ag = _dot(am_ref[...], wo_ref[...], 1, 1) + bo_ref[...]
    mg = _dot(mm_ref[...], wo_ref[...], 1, 1) + bo_ref[...]
    c = jnp.concatenate([ag, mg], axis=1)
    h = jnp.maximum(_dot(c, w1_ref[...]) + b1_ref[...], 0.0)
    o_ref[...] = _dot(h, w2_ref[...]) + b2_ref[...]


def _final(atom_mean, motif_mean, wo, bo, w1, b1, w2, b2):
    return pl.pallas_call(
        _final_body,
        out_shape=jax.ShapeDtypeStruct((1, 128), jnp.float32),
    )(atom_mean, motif_mean, wo, bo.reshape(1, HIDDEN),
      w1, b1.reshape(1, HIDDEN), w2, b2.reshape(1, 128))


def _build_adjacency(edge_index, motif_edge_index):
    flat, vals, mflat = _flat_ids(edge_index, motif_edge_index)
    a_flat, am_flat = _sc_adjacency_kernel()(flat, vals, mflat)
    return (a_flat.reshape(N_ATOM, PCOL),
            am_flat.reshape(N_MOTIF, N_MOTIF))


def kernel(atom_features, bond_features, motif_features, params,
           edge_index, motif_edge_index):
    del bond_features
    p = params
    A, Am = _build_adjacency(edge_index, motif_edge_index)

    dinv, hd = _gcn_pre(A, atom_features, p['atom_W'], p['atom_b'],
                        p['gcn_W'][0])
    hd = _gcn_fused(A, hd, dinv, p['gcn_b'][0], p['gcn_W'][1])
    hd = _gcn_fused(A, hd, dinv, p['gcn_b'][1], p['gcn_W'][2])
    x = _gcn_fused(A, hd, dinv, p['gcn_b'][2], None)

    mm = _motif_pooled(Am, motif_features, p['motif_W'], p['motif_b'],
                       p['gin_W1'], p['gin_b1'], p['gin_W2'], p['gin_b2'],
                       p['attn_Wqkv'], p['attn_bqkv'])
    am = _attn_pool(x, p['attn_Wqkv'], p['attn_bqkv']).reshape(1, HIDDEN)

    latent = _final(am, mm, p['attn_Wo'], p['attn_bo'],
                    p['proj_W1'], p['proj_b1'], p['proj_W2'], p['proj_b2'])
    return latent.reshape(128)


# speedup vs baseline: 16.3234x; 1.0161x over previous
import functools

import jax
import jax.numpy as jnp
from jax import lax
from jax.experimental import pallas as pl
from jax.experimental.pallas import tpu as pltpu
from jax.experimental.pallas import tpu_sc as plsc

N_ATOM = 2048
E_ATOM = 65536
N_MOTIF = 512
E_MOTIF = 2048
HIDDEN = 256
HEADS = 4
HD = HIDDEN // HEADS

NW = 32
CH = 64
PCOL = N_ATOM // 2
CWORDS = CH * PCOL
EBLK = 4096
MCH = N_MOTIF // NW

RB = 256
NRB = N_ATOM // RB

_PREC = jax.lax.Precision.DEFAULT


@functools.cache
def _sc_adjacency_kernel():
    mesh = plsc.VectorSubcoreMesh(core_axis_name="c", subcore_axis_name="s")
    return pl.kernel(
        _sc_adjacency_body,
        mesh=mesh,
        compiler_params=pltpu.CompilerParams(needs_layout_passes=False),
        out_type=[
            jax.ShapeDtypeStruct((N_ATOM, PCOL), jnp.int32),
            jax.ShapeDtypeStruct((N_MOTIF, N_MOTIF), jnp.float32),
        ],
        scratch_types=[
            pltpu.VMEM((CH, PCOL), jnp.int32),
            pltpu.VMEM((EBLK,), jnp.int32),
            pltpu.VMEM((EBLK,), jnp.int32),
            pltpu.VMEM((EBLK,), jnp.int32),
            pltpu.VMEM((EBLK,), jnp.int32),
            pltpu.VMEM((MCH, N_MOTIF), jnp.float32),
            pltpu.VMEM((E_MOTIF,), jnp.int32),
            pltpu.SemaphoreType.DMA,
            pltpu.SemaphoreType.DMA,
        ],
    )


def _sc_adjacency_body(flat_hbm, val_hbm, mflat_hbm, a_out, am_out,
                       buf, fbuf0, fbuf1, vbuf0, vbuf1, mbuf, mfbuf,
                       sem0, sem1):
    wid = lax.axis_index("s") * 2 + lax.axis_index("c")
    ones = jnp.ones((16,), jnp.float32)
    zeros = jnp.zeros((16,), jnp.float32)
    izeros = jnp.zeros((16,), jnp.int32)
    fbase = wid * CWORDS
    fbufs, vbufs, sems = (fbuf0, fbuf1), (vbuf0, vbuf1), (sem0, sem1)
    nblk = E_ATOM // EBLK

    def _start(blk):
        slot = blk % 2
        return (
            pltpu.async_copy(flat_hbm.at[pl.ds(blk * EBLK, EBLK)],
                             fbufs[slot], sems[slot]),
            pltpu.async_copy(val_hbm.at[pl.ds(blk * EBLK, EBLK)],
                             vbufs[slot], sems[slot]),
        )

    pending = _start(0)

    def zbody(i, _):
        r = i >> 3
        cb = (i & 7) * 128
        for k in range(8):
            buf[r, pl.ds(cb + k * 16, 16)] = izeros
        return _
    lax.fori_loop(0, CWORDS // 128, zbody, 0)

    for blk in range(nblk):
        slot = blk % 2
        for h in pending:
            h.wait()
        if blk + 1 < nblk:
            pending = _start(blk + 1)
        fbuf, vbuf = fbufs[slot], vbufs[slot]

        def ebody(i, _):
            for k in range(4):
                off = i * 64 + k * 16
                rel = fbuf[pl.ds(off, 16)] - fbase
                m = rel.astype(jnp.uint32) < CWORDS
                plsc.addupdate_scatter(
                    buf, [rel >> 10, rel & (PCOL - 1)],
                    vbuf[pl.ds(off, 16)], mask=m)
            return _
        lax.fori_loop(0, EBLK // 64, ebody, 0)

    pltpu.sync_copy(buf, a_out.at[pl.ds(wid * CH, CH)])

    mwords = MCH * N_MOTIF
    mfbase = wid * mwords
    pltpu.sync_copy(mflat_hbm, mfbuf)

    def mzbody(i, _):
        r = i >> 2
        cb = (i & 3) * 128
        for k in range(8):
            mbuf[r, pl.ds(cb + k * 16, 16)] = zeros
        return _
    lax.fori_loop(0, mwords // 128, mzbody, 0)

    def mebody(i, _):
        for k in range(4):
            rel = mfbuf[pl.ds(i * 64 + k * 16, 16)] - mfbase
            m = rel.astype(jnp.uint32) < mwords
            plsc.addupdate_scatter(mbuf, [rel >> 9, rel & (N_MOTIF - 1)],
                                   ones, mask=m)
        return _
    lax.fori_loop(0, E_MOTIF // 64, mebody, 0)

    pltpu.sync_copy(mbuf, am_out.at[pl.ds(wid * MCH, MCH)])


def _dot(a, b, ca=1, cb=0):
    return lax.dot_general(a, b, (((ca,), (cb,)), ((), ())), precision=_PREC)


def _full(shape):
    return pl.BlockSpec(shape, lambda *_: (0,) * len(shape))


def _flat_body(e_ref, me_ref, f_ref, v_ref, mf_ref):
    src, dst = e_ref[0], e_ref[1]
    f_ref[...] = dst * PCOL + (src & (PCOL - 1))
    v_ref[...] = 1 << ((src >> 10) << 4)
    mf_ref[...] = me_ref[1] * N_MOTIF + me_ref[0]


def _flat_ids(edge_index, motif_edge_index):
    f, v, mf = pl.pallas_call(
        _flat_body,
        out_shape=[
            jax.ShapeDtypeStruct((E_ATOM // 128, 128), jnp.int32),
            jax.ShapeDtypeStruct((E_ATOM // 128, 128), jnp.int32),
            jax.ShapeDtypeStruct((E_MOTIF // 128, 128), jnp.int32),
        ],
    )(edge_index.reshape(2, E_ATOM // 128, 128),
      motif_edge_index.reshape(2, E_MOTIF // 128, 128))
    return f.reshape(E_ATOM), v.reshape(E_ATOM), mf.reshape(E_MOTIF)


def _gcn_pre_body(a_ref, xf_ref, aw_ref, ab_ref, w0_ref,
                  abf_ref, dinv_ref, hd_ref, hdb_ref):
    ap = a_ref[...]
    low = ap & 0xFFFF
    high = ap >> 16
    abf_ref[:, :PCOL] = low.astype(jnp.bfloat16)
    abf_ref[:, PCOL:] = high.astype(jnp.bfloat16)
    deg = (jnp.sum(low, axis=1, keepdims=True)
           + jnp.sum(high, axis=1, keepdims=True)).astype(jnp.float32) + 1.0
    dinv = lax.rsqrt(deg)
    dinv_ref[...] = dinv
    x0 = _dot(xf_ref[...], aw_ref[...]) + ab_ref[...]
    hd = dinv * _dot(x0, w0_ref[...])
    hd_ref[...] = hd
    hdb_ref[...] = hd.astype(jnp.bfloat16)


def _gcn_pre(A, atom_f, aw, ab, w0):
    return pl.pallas_call(
        _gcn_pre_body,
        grid=(NRB,),
        in_specs=[pl.BlockSpec((RB, PCOL), lambda i: (i, 0)),
                  pl.BlockSpec((RB, 128), lambda i: (i, 0)),
                  _full((128, HIDDEN)), _full((1, HIDDEN)),
                  _full((HIDDEN, HIDDEN))],
        out_specs=[pl.BlockSpec((RB, N_ATOM), lambda i: (i, 0)),
                   pl.BlockSpec((RB, 1), lambda i: (i, 0)),
                   pl.BlockSpec((RB, HIDDEN), lambda i: (i, 0)),
                   pl.BlockSpec((RB, HIDDEN), lambda i: (i, 0))],
        out_shape=[jax.ShapeDtypeStruct((N_ATOM, N_ATOM), jnp.bfloat16),
                   jax.ShapeDtypeStruct((N_ATOM, 1), jnp.float32),
                   jax.ShapeDtypeStruct((N_ATOM, HIDDEN), jnp.float32),
                   jax.ShapeDtypeStruct((N_ATOM, HIDDEN), jnp.bfloat16)],
    )(A, atom_f, aw, ab.reshape(1, HIDDEN), w0)


def _gcn_agg(abf_ref, hdbf_ref, hd_ref):
    t = lax.dot_general(abf_ref[...], hdbf_ref[...],
                        (((1,), (0,)), ((), ())),
                        preferred_element_type=jnp.float32)
    return t + hd_ref[...]


def _gcn_fused_mid(abf_ref, hdbf_ref, hd_ref, dinv_ref, b_ref,
                   wn_ref, o_ref, ob_ref):
    t = _gcn_agg(abf_ref, hdbf_ref, hd_ref)
    x = jnp.maximum(dinv_ref[...] * t + b_ref[...], 0.0)
    hd = dinv_ref[...] * _dot(x, wn_ref[...])
    o_ref[...] = hd
    ob_ref[...] = hd.astype(jnp.bfloat16)


def _gcn_fused_last(abf_ref, hdbf_ref, hd_ref, dinv_ref, b_ref, o_ref):
    t = _gcn_agg(abf_ref, hdbf_ref, hd_ref)
    o_ref[...] = jnp.maximum(dinv_ref[...] * t + b_ref[...], 0.0)


def _gcn_fused(Abf, hd, hdb, dinv, b, w_next):
    specs = [pl.BlockSpec((RB, N_ATOM), lambda i: (i, 0)),
             _full((N_ATOM, HIDDEN)),
             pl.BlockSpec((RB, HIDDEN), lambda i: (i, 0)),
             pl.BlockSpec((RB, 1), lambda i: (i, 0)),
             _full((1, HIDDEN))]
    args = [Abf, hdb, hd, dinv, b.reshape(1, HIDDEN)]
    blk = pl.BlockSpec((RB, HIDDEN), lambda i: (i, 0))
    if w_next is None:
        return pl.pallas_call(
            _gcn_fused_last,
            grid=(NRB,),
            in_specs=specs,
            out_specs=blk,
            out_shape=jax.ShapeDtypeStruct((N_ATOM, HIDDEN), jnp.float32),
        )(*args)
    specs.append(_full((HIDDEN, HIDDEN)))
    args.append(w_next)
    return pl.pallas_call(
        _gcn_fused_mid,
        grid=(NRB,),
        in_specs=specs,
        out_specs=[blk, blk],
        out_shape=[jax.ShapeDtypeStruct((N_ATOM, HIDDEN), jnp.float32),
                   jax.ShapeDtypeStruct((N_ATOM, HIDDEN), jnp.bfloat16)],
    )(*args)


def _softmax_rows(s):
    s = s - jnp.max(s, axis=1, keepdims=True)
    e = jnp.exp(s)
    return e / jnp.sum(e, axis=1, keepdims=True)


def _motif_body(am_ref, mf_ref, mw_ref, mb_ref,
                w1_ref, b1_ref, w2_ref, b2_ref,
                wqkv_ref, bqkv_ref, o_ref):
    m = _dot(mf_ref[...], mw_ref[...]) + mb_ref[...]
    Am = am_ref[...]
    for i in range(3):
        h = m + _dot(Am, m)
        h1 = jnp.maximum(_dot(h, w1_ref[i]) + b1_ref[i][None, :], 0.0)
        m = jnp.maximum(_dot(h1, w2_ref[i]) + b2_ref[i][None, :], 0.0)
    qkv = _dot(m, wqkv_ref[...], 1, 1) + bqkv_ref[...]
    outs = []
    for h in range(HEADS):
        q = qkv[:, h * HD:(h + 1) * HD]
        k = qkv[:, HIDDEN + h * HD:HIDDEN + (h + 1) * HD]
        v = qkv[:, 2 * HIDDEN + h * HD:2 * HIDDEN + (h + 1) * HD]
        s = lax.dot_general(q, k, (((1,), (1,)), ((), ())),
                            precision=_PREC) * (1.0 / float(HD) ** 0.5)
        p = _softmax_rows(s)
        o = _dot(p, v)
        outs.append(jnp.sum(o, axis=0, keepdims=True) * (1.0 / N_MOTIF))
    o_ref[...] = jnp.concatenate(outs, axis=1)


def _motif_pooled(Am, motif_f, mw, mb, w1, b1, w2, b2, wqkv, bqkv):
    return pl.pallas_call(
        _motif_body,
        out_shape=jax.ShapeDtypeStruct((1, HIDDEN), jnp.float32),
    )(Am, motif_f, mw, mb.reshape(1, HIDDEN), w1, b1, w2, b2,
      wqkv, bqkv.reshape(1, 3 * HIDDEN))


def _attn_pool_body(xb_ref, xf_ref, wq_ref, wk_ref, wv_ref,
                    bq_ref, bk_ref, bv_ref, o_ref, k_s, v_s):
    j = pl.program_id(1)
    L = xf_ref.shape[0]

    @pl.when(j == 0)
    def _():
        k_s[...] = _dot(xf_ref[...], wk_ref[...], 1, 1) + bk_ref[0]
        v_s[...] = _dot(xf_ref[...], wv_ref[...], 1, 1) + bv_ref[0]

    q = _dot(xb_ref[...], wq_ref[...], 1, 1) + bq_ref[0]
    s = lax.dot_general(q, k_s[...], (((1,), (1,)), ((), ())),
                        precision=_PREC) * (1.0 / float(HD) ** 0.5)
    p = _softmax_rows(s)
    o = _dot(p, v_s[...])
    colsum = jnp.sum(o, axis=0, keepdims=True) * (1.0 / float(L))

    @pl.when(j == 0)
    def _():
        o_ref[...] = jnp.zeros_like(o_ref)

    o_ref[0] += colsum


def _attn_pool(x, wqkv, bqkv):
    L = x.shape[0]
    b3 = bqkv.reshape(3 * HEADS, 1, HD)
    return pl.pallas_call(
        _attn_pool_body,
        grid=(HEADS, L // RB),
        in_specs=[
            pl.BlockSpec((RB, HIDDEN), lambda h, j: (j, 0)),
            _full((L, HIDDEN)),
            pl.BlockSpec((HD, HIDDEN), lambda h, j: (h, 0)),
            pl.BlockSpec((HD, HIDDEN), lambda h, j: (HEADS + h, 0)),
            pl.BlockSpec((HD, HIDDEN), lambda h, j: (2 * HEADS + h, 0)),
            pl.BlockSpec((1, 1, HD), lambda h, j: (h, 0, 0)),
            pl.BlockSpec((1, 1, HD), lambda h, j: (HEADS + h, 0, 0)),
            pl.BlockSpec((1, 1, HD), lambda h, j: (2 * HEADS + h, 0, 0)),
        ],
        out_specs=pl.BlockSpec((1, 1, HD), lambda h, j: (h, 0, 0)),
        out_shape=jax.ShapeDtypeStruct((HEADS, 1, HD), jnp.float32),
        scratch_shapes=[pltpu.VMEM((L, HD), jnp.float32),
                        pltpu.VMEM((L, HD), jnp.float32)],
    )(x, x, wqkv, wqkv, wqkv, b3, b3, b3)


def _final_body(am_ref, mm_ref, wo_ref, bo_ref,
                w1_ref, b1_ref, w2_ref, b2_ref, o_ref):
    ag = _dot(am_ref[...], wo_ref[...], 1, 1) + bo_ref[...]
    mg = _dot(mm_ref[...], wo_ref[...], 1, 1) + bo_ref[...]
    c = jnp.concatenate([ag, mg], axis=1)
    h = jnp.maximum(_dot(c, w1_ref[...]) + b1_ref[...], 0.0)
    o_ref[...] = _dot(h, w2_ref[...]) + b2_ref[...]


def _final(atom_mean, motif_mean, wo, bo, w1, b1, w2, b2):
    return pl.pallas_call(
        _final_body,
        out_shape=jax.ShapeDtypeStruct((1, 128), jnp.float32),
    )(atom_mean, motif_mean, wo, bo.reshape(1, HIDDEN),
      w1, b1.reshape(1, HIDDEN), w2, b2.reshape(1, 128))


def _build_adjacency(edge_index, motif_edge_index):
    flat, vals, mflat = _flat_ids(edge_index, motif_edge_index)
    return _sc_adjacency_kernel()(flat, vals, mflat)


def kernel(atom_features, bond_features, motif_features, params,
           edge_index, motif_edge_index):
    del bond_features
    p = params
    A, Am = _build_adjacency(edge_index, motif_edge_index)

    Abf, dinv, hd, hdb = _gcn_pre(A, atom_features, p['atom_W'],
                                  p['atom_b'], p['gcn_W'][0])
    hd, hdb = _gcn_fused(Abf, hd, hdb, dinv, p['gcn_b'][0], p['gcn_W'][1])
    hd, hdb = _gcn_fused(Abf, hd, hdb, dinv, p['gcn_b'][1], p['gcn_W'][2])
    x = _gcn_fused(Abf, hd, hdb, dinv, p['gcn_b'][2], None)

    mm = _motif_pooled(Am, motif_features, p['motif_W'], p['motif_b'],
                       p['gin_W1'], p['gin_b1'], p['gin_W2'], p['gin_b2'],
                       p['attn_Wqkv'], p['attn_bqkv'])
    am = _attn_pool(x, p['attn_Wqkv'], p['attn_bqkv']).reshape(1, HIDDEN)

    latent = _final(am, mm, p['attn_Wo'], p['attn_bo'],
                    p['proj_W1'], p['proj_b1'], p['proj_W2'], p['proj_b2'])
    return latent.reshape(128)


# speedup vs baseline: 18.1999x; 1.1150x over previous
import functools

import jax
import jax.numpy as jnp
from jax import lax
from jax.experimental import pallas as pl
from jax.experimental.pallas import tpu as pltpu
from jax.experimental.pallas import tpu_sc as plsc

N_ATOM = 2048
E_ATOM = 65536
N_MOTIF = 512
E_MOTIF = 2048
HIDDEN = 256
HEADS = 4
HD = HIDDEN // HEADS

NW = 32
CH = 64
PCOL = N_ATOM // 2
CWORDS = CH * PCOL
EBLK = 4096
MCH = N_MOTIF // NW

RB = 256
NRB = N_ATOM // RB

_PREC = jax.lax.Precision.DEFAULT


@functools.cache
def _sc_adjacency_kernel():
    mesh = plsc.VectorSubcoreMesh(core_axis_name="c", subcore_axis_name="s")
    return pl.kernel(
        _sc_adjacency_body,
        mesh=mesh,
        compiler_params=pltpu.CompilerParams(needs_layout_passes=False),
        out_type=[
            jax.ShapeDtypeStruct((N_ATOM, PCOL), jnp.int32),
            jax.ShapeDtypeStruct((N_MOTIF, N_MOTIF), jnp.float32),
        ],
        scratch_types=[
            pltpu.VMEM((CH, PCOL), jnp.int32),
            pltpu.VMEM((EBLK,), jnp.int32),
            pltpu.VMEM((EBLK,), jnp.int32),
            pltpu.VMEM((EBLK,), jnp.int32),
            pltpu.VMEM((EBLK,), jnp.int32),
            pltpu.VMEM((MCH, N_MOTIF), jnp.float32),
            pltpu.VMEM((E_MOTIF,), jnp.int32),
            pltpu.SemaphoreType.DMA,
            pltpu.SemaphoreType.DMA,
        ],
    )


def _sc_adjacency_body(flat_hbm, val_hbm, mflat_hbm, a_out, am_out,
                       buf, fbuf0, fbuf1, vbuf0, vbuf1, mbuf, mfbuf,
                       sem0, sem1):
    wid = lax.axis_index("s") * 2 + lax.axis_index("c")
    ones = jnp.ones((16,), jnp.float32)
    zeros = jnp.zeros((16,), jnp.float32)
    izeros = jnp.zeros((16,), jnp.int32)
    fbase = wid * CWORDS
    fbufs, vbufs, sems = (fbuf0, fbuf1), (vbuf0, vbuf1), (sem0, sem1)
    nblk = E_ATOM // EBLK

    def _start(blk):
        slot = blk % 2
        return (
            pltpu.async_copy(flat_hbm.at[pl.ds(blk * EBLK, EBLK)],
                             fbufs[slot], sems[slot]),
            pltpu.async_copy(val_hbm.at[pl.ds(blk * EBLK, EBLK)],
                             vbufs[slot], sems[slot]),
        )

    pending = _start(0)

    def zbody(i, _):
        r = i >> 3
        cb = (i & 7) * 128
        for k in range(8):
            buf[r, pl.ds(cb + k * 16, 16)] = izeros
        return _
    lax.fori_loop(0, CWORDS // 128, zbody, 0)

    for blk in range(nblk):
        slot = blk % 2
        for h in pending:
            h.wait()
        if blk + 1 < nblk:
            pending = _start(blk + 1)
        fbuf, vbuf = fbufs[slot], vbufs[slot]

        def ebody(i, _):
            for k in range(4):
                off = i * 64 + k * 16
                rel = fbuf[pl.ds(off, 16)] - fbase
                m = rel.astype(jnp.uint32) < CWORDS
                plsc.addupdate_scatter(
                    buf, [rel >> 10, rel & (PCOL - 1)],
                    vbuf[pl.ds(off, 16)], mask=m)
            return _
        lax.fori_loop(0, EBLK // 64, ebody, 0)

    pltpu.sync_copy(buf, a_out.at[pl.ds(wid * CH, CH)])

    mwords = MCH * N_MOTIF
    mfbase = wid * mwords
    pltpu.sync_copy(mflat_hbm, mfbuf)

    def mzbody(i, _):
        r = i >> 2
        cb = (i & 3) * 128
        for k in range(8):
            mbuf[r, pl.ds(cb + k * 16, 16)] = zeros
        return _
    lax.fori_loop(0, mwords // 128, mzbody, 0)

    def mebody(i, _):
        for k in range(4):
            rel = mfbuf[pl.ds(i * 64 + k * 16, 16)] - mfbase
            m = rel.astype(jnp.uint32) < mwords
            plsc.addupdate_scatter(mbuf, [rel >> 9, rel & (N_MOTIF - 1)],
                                   ones, mask=m)
        return _
    lax.fori_loop(0, E_MOTIF // 64, mebody, 0)

    pltpu.sync_copy(mbuf, am_out.at[pl.ds(wid * MCH, MCH)])


def _dot(a, b, ca=1, cb=0):
    return lax.dot_general(a, b, (((ca,), (cb,)), ((), ())), precision=_PREC)


def _full(shape):
    return pl.BlockSpec(shape, lambda *_: (0,) * len(shape))


def _flat_body(e_ref, me_ref, f_ref, v_ref, mf_ref):
    src, dst = e_ref[0], e_ref[1]
    f_ref[...] = dst * PCOL + (src & (PCOL - 1))
    v_ref[...] = 1 << ((src >> 10) << 4)
    mf_ref[...] = me_ref[1] * N_MOTIF + me_ref[0]


def _flat_ids(edge_index, motif_edge_index):
    f, v, mf = pl.pallas_call(
        _flat_body,
        out_shape=[
            jax.ShapeDtypeStruct((E_ATOM // 128, 128), jnp.int32),
            jax.ShapeDtypeStruct((E_ATOM // 128, 128), jnp.int32),
            jax.ShapeDtypeStruct((E_MOTIF // 128, 128), jnp.int32),
        ],
    )(edge_index.reshape(2, E_ATOM // 128, 128),
      motif_edge_index.reshape(2, E_MOTIF // 128, 128))
    return f.reshape(E_ATOM), v.reshape(E_ATOM), mf.reshape(E_MOTIF)


def _gcn_pre_body(a_ref, xf_ref, aw_ref, ab_ref, w0_ref,
                  abf_ref, dinv_ref, hd_ref, hdb_ref):
    ap = a_ref[...]
    low = ap & 0xFFFF
    high = ap >> 16
    abf_ref[:, :PCOL] = low.astype(jnp.bfloat16)
    abf_ref[:, PCOL:] = high.astype(jnp.bfloat16)
    deg = (jnp.sum(low, axis=1, keepdims=True)
           + jnp.sum(high, axis=1, keepdims=True)).astype(jnp.float32) + 1.0
    dinv = lax.rsqrt(deg)
    dinv_ref[...] = dinv
    x0 = _dot(xf_ref[...], aw_ref[...]) + ab_ref[...]
    hd = dinv * _dot(x0, w0_ref[...])
    hd_ref[...] = hd
    hdb_ref[...] = hd.astype(jnp.bfloat16)


def _gcn_pre(A, atom_f, aw, ab, w0):
    return pl.pallas_call(
        _gcn_pre_body,
        grid=(NRB,),
        in_specs=[pl.BlockSpec((RB, PCOL), lambda i: (i, 0)),
                  pl.BlockSpec((RB, 128), lambda i: (i, 0)),
                  _full((128, HIDDEN)), _full((1, HIDDEN)),
                  _full((HIDDEN, HIDDEN))],
        out_specs=[pl.BlockSpec((RB, N_ATOM), lambda i: (i, 0)),
                   pl.BlockSpec((RB, 1), lambda i: (i, 0)),
                   pl.BlockSpec((RB, HIDDEN), lambda i: (i, 0)),
                   pl.BlockSpec((RB, HIDDEN), lambda i: (i, 0))],
        out_shape=[jax.ShapeDtypeStruct((N_ATOM, N_ATOM), jnp.bfloat16),
                   jax.ShapeDtypeStruct((N_ATOM, 1), jnp.float32),
                   jax.ShapeDtypeStruct((N_ATOM, HIDDEN), jnp.float32),
                   jax.ShapeDtypeStruct((N_ATOM, HIDDEN), jnp.bfloat16)],
    )(A, atom_f, aw, ab.reshape(1, HIDDEN), w0)


def _gcn_agg(abf_ref, hdbf_ref, hd_ref):
    t = lax.dot_general(abf_ref[...], hdbf_ref[...],
                        (((1,), (0,)), ((), ())),
                        preferred_element_type=jnp.float32)
    return t + hd_ref[...]


def _gcn_fused_mid(abf_ref, hdbf_ref, hd_ref, dinv_ref, b_ref,
                   wn_ref, o_ref, ob_ref):
    t = _gcn_agg(abf_ref, hdbf_ref, hd_ref)
    x = jnp.maximum(dinv_ref[...] * t + b_ref[...], 0.0)
    hd = dinv_ref[...] * _dot(x, wn_ref[...])
    o_ref[...] = hd
    ob_ref[...] = hd.astype(jnp.bfloat16)


def _gcn_fused_last(abf_ref, hdbf_ref, hd_ref, dinv_ref, b_ref, o_ref):
    t = _gcn_agg(abf_ref, hdbf_ref, hd_ref)
    o_ref[...] = jnp.maximum(dinv_ref[...] * t + b_ref[...], 0.0)


def _gcn_fused(Abf, hd, hdb, dinv, b, w_next):
    specs = [pl.BlockSpec((RB, N_ATOM), lambda i: (i, 0)),
             _full((N_ATOM, HIDDEN)),
             pl.BlockSpec((RB, HIDDEN), lambda i: (i, 0)),
             pl.BlockSpec((RB, 1), lambda i: (i, 0)),
             _full((1, HIDDEN))]
    args = [Abf, hdb, hd, dinv, b.reshape(1, HIDDEN)]
    blk = pl.BlockSpec((RB, HIDDEN), lambda i: (i, 0))
    if w_next is None:
        return pl.pallas_call(
            _gcn_fused_last,
            grid=(NRB,),
            in_specs=specs,
            out_specs=blk,
            out_shape=jax.ShapeDtypeStruct((N_ATOM, HIDDEN), jnp.float32),
        )(*args)
    specs.append(_full((HIDDEN, HIDDEN)))
    args.append(w_next)
    return pl.pallas_call(
        _gcn_fused_mid,
        grid=(NRB,),
        in_specs=specs,
        out_specs=[blk, blk],
        out_shape=[jax.ShapeDtypeStruct((N_ATOM, HIDDEN), jnp.float32),
                   jax.ShapeDtypeStruct((N_ATOM, HIDDEN), jnp.bfloat16)],
    )(*args)


def _softmax_exp(s):
    return jnp.exp((s - jnp.max(s, axis=1, keepdims=True))
                   .astype(jnp.bfloat16))


def _motif_body(am_ref, mf_ref, mw_ref, mb_ref,
                w1_ref, b1_ref, w2_ref, b2_ref,
                wqkv_ref, bqkv_ref, o_ref):
    m = _dot(mf_ref[...], mw_ref[...]) + mb_ref[...]
    Am = am_ref[...]
    for i in range(3):
        h = m + _dot(Am, m)
        h1 = jnp.maximum(_dot(h, w1_ref[i]) + b1_ref[i][None, :], 0.0)
        m = jnp.maximum(_dot(h1, w2_ref[i]) + b2_ref[i][None, :], 0.0)
    qkv = _dot(m, wqkv_ref[...], 1, 1) + bqkv_ref[...]
    outs = []
    scale = 1.0 / float(HD) ** 0.5
    for h in range(HEADS):
        q = qkv[:, h * HD:(h + 1) * HD] * scale
        k = qkv[:, HIDDEN + h * HD:HIDDEN + (h + 1) * HD]
        v = qkv[:, 2 * HIDDEN + h * HD:2 * HIDDEN + (h + 1) * HD]
        s = lax.dot_general(q, k, (((1,), (1,)), ((), ())),
                            precision=_PREC)
        e = _softmax_exp(s)
        vx = jnp.concatenate(
            [v, jnp.ones((N_MOTIF, 16), jnp.float32)], axis=1)
        ox = lax.dot_general(e, vx.astype(jnp.bfloat16),
                             (((1,), (0,)), ((), ())),
                             preferred_element_type=jnp.float32)
        o = ox[:, :HD] * (1.0 / ox[:, HD:HD + 1])
        outs.append(jnp.sum(o, axis=0, keepdims=True) * (1.0 / N_MOTIF))
    o_ref[...] = jnp.concatenate(outs, axis=1)


def _motif_pooled(Am, motif_f, mw, mb, w1, b1, w2, b2, wqkv, bqkv):
    return pl.pallas_call(
        _motif_body,
        out_shape=jax.ShapeDtypeStruct((1, HIDDEN), jnp.float32),
    )(Am, motif_f, mw, mb.reshape(1, HIDDEN), w1, b1, w2, b2,
      wqkv, bqkv.reshape(1, 3 * HIDDEN))


def _attn_pool_body(xb_ref, xf_ref, wq_ref, wk_ref, wv_ref,
                    bq_ref, bk_ref, bv_ref, o_ref, k_s, v_s):
    j = pl.program_id(1)
    L = xf_ref.shape[0]

    @pl.when(j == 0)
    def _():
        k_s[...] = _dot(xf_ref[...], wk_ref[...], 1, 1) + bk_ref[0]
        v = _dot(xf_ref[...], wv_ref[...], 1, 1) + bv_ref[0]
        v_s[:, :HD] = v.astype(jnp.bfloat16)
        v_s[:, HD:HD + 16] = jnp.ones((L, 16), jnp.bfloat16)

    q = (_dot(xb_ref[...], wq_ref[...], 1, 1) + bq_ref[0]) \
        * (1.0 / float(HD) ** 0.5)
    s = lax.dot_general(q, k_s[...], (((1,), (1,)), ((), ())),
                        precision=_PREC)
    e = _softmax_exp(s)
    ox = lax.dot_general(e, v_s[...], (((1,), (0,)), ((), ())),
                         preferred_element_type=jnp.float32)
    o = ox[:, :HD] * (1.0 / ox[:, HD:HD + 1])
    colsum = jnp.sum(o, axis=0, keepdims=True) * (1.0 / float(L))

    @pl.when(j == 0)
    def _():
        o_ref[...] = jnp.zeros_like(o_ref)

    o_ref[0] += colsum


def _attn_pool(x, wqkv, bqkv):
    L = x.shape[0]
    b3 = bqkv.reshape(3 * HEADS, 1, HD)
    return pl.pallas_call(
        _attn_pool_body,
        grid=(HEADS, L // RB),
        in_specs=[
            pl.BlockSpec((RB, HIDDEN), lambda h, j: (j, 0)),
            _full((L, HIDDEN)),
            pl.BlockSpec((HD, HIDDEN), lambda h, j: (h, 0)),
            pl.BlockSpec((HD, HIDDEN), lambda h, j: (HEADS + h, 0)),
            pl.BlockSpec((HD, HIDDEN), lambda h, j: (2 * HEADS + h, 0)),
            pl.BlockSpec((1, 1, HD), lambda h, j: (h, 0, 0)),
            pl.BlockSpec((1, 1, HD), lambda h, j: (HEADS + h, 0, 0)),
            pl.BlockSpec((1, 1, HD), lambda h, j: (2 * HEADS + h, 0, 0)),
        ],
        out_specs=pl.BlockSpec((1, 1, HD), lambda h, j: (h, 0, 0)),
        out_shape=jax.ShapeDtypeStruct((HEADS, 1, HD), jnp.float32),
        scratch_shapes=[pltpu.VMEM((L, HD), jnp.float32),
                        pltpu.VMEM((L, HD + 16), jnp.bfloat16)],
    )(x, x, wqkv, wqkv, wqkv, b3, b3, b3)


def _final_body(am_ref, mm_ref, wo_ref, bo_ref,
                w1_ref, b1_ref, w2_ref, b2_ref, o_ref):
    ag = _dot(am_ref[...], wo_ref[...], 1, 1) + bo_ref[...]
    mg = _dot(mm_ref[...], wo_ref[...], 1, 1) + bo_ref[...]
    c = jnp.concatenate([ag, mg], axis=1)
    h = jnp.maximum(_dot(c, w1_ref[...]) + b1_ref[...], 0.0)
    o_ref[...] = _dot(h, w2_ref[...]) + b2_ref[...]


def _final(atom_mean, motif_mean, wo, bo, w1, b1, w2, b2):
    return pl.pallas_call(
        _final_body,
        out_shape=jax.ShapeDtypeStruct((1, 128), jnp.float32),
    )(atom_mean, motif_mean, wo, bo.reshape(1, HIDDEN),
      w1, b1.reshape(1, HIDDEN), w2, b2.reshape(1, 128))


def _build_adjacency(edge_index, motif_edge_index):
    flat, vals, mflat = _flat_ids(edge_index, motif_edge_index)
    return _sc_adjacency_kernel()(flat, vals, mflat)


def kernel(atom_features, bond_features, motif_features, params,
           edge_index, motif_edge_index):
    del bond_features
    p = params
    A, Am = _build_adjacency(edge_index, motif_edge_index)

    Abf, dinv, hd, hdb = _gcn_pre(A, atom_features, p['atom_W'],
                                  p['atom_b'], p['gcn_W'][0])
    hd, hdb = _gcn_fused(Abf, hd, hdb, dinv, p['gcn_b'][0], p['gcn_W'][1])
    hd, hdb = _gcn_fused(Abf, hd, hdb, dinv, p['gcn_b'][1], p['gcn_W'][2])
    x = _gcn_fused(Abf, hd, hdb, dinv, p['gcn_b'][2], None)

    mm = _motif_pooled(Am, motif_features, p['motif_W'], p['motif_b'],
                       p['gin_W1'], p['gin_b1'], p['gin_W2'], p['gin_b2'],
                       p['attn_Wqkv'], p['attn_bqkv'])
    am = _attn_pool(x, p['attn_Wqkv'], p['attn_bqkv']).reshape(1, HIDDEN)

    latent = _final(am, mm, p['attn_Wo'], p['attn_bo'],
                    p['proj_W1'], p['proj_b1'], p['proj_W2'], p['proj_b2'])
    return latent.reshape(128)


# speedup vs baseline: 18.7106x; 1.0281x over previous
import functools

import jax
import jax.numpy as jnp
from jax import lax
from jax.experimental import pallas as pl
from jax.experimental.pallas import tpu as pltpu
from jax.experimental.pallas import tpu_sc as plsc

N_ATOM = 2048
E_ATOM = 65536
N_MOTIF = 512
E_MOTIF = 2048
HIDDEN = 256
HEADS = 4
HD = HIDDEN // HEADS

NW = 32
CH = 64
PCOL = N_ATOM // 2
CWORDS = CH * PCOL
EBLK = 4096
MCH = N_MOTIF // NW

RB = 512
NRB = N_ATOM // RB

_PREC = jax.lax.Precision.DEFAULT


@functools.cache
def _sc_adjacency_kernel():
    mesh = plsc.VectorSubcoreMesh(core_axis_name="c", subcore_axis_name="s")
    return pl.kernel(
        _sc_adjacency_body,
        mesh=mesh,
        compiler_params=pltpu.CompilerParams(needs_layout_passes=False),
        out_type=[
            jax.ShapeDtypeStruct((N_ATOM, PCOL), jnp.int32),
            jax.ShapeDtypeStruct((N_MOTIF, N_MOTIF), jnp.float32),
        ],
        scratch_types=[
            pltpu.VMEM((CH, PCOL), jnp.int32),
            pltpu.VMEM((EBLK,), jnp.int32),
            pltpu.VMEM((EBLK,), jnp.int32),
            pltpu.VMEM((EBLK,), jnp.int32),
            pltpu.VMEM((EBLK,), jnp.int32),
            pltpu.VMEM((MCH, N_MOTIF), jnp.float32),
            pltpu.VMEM((E_MOTIF,), jnp.int32),
            pltpu.SemaphoreType.DMA,
            pltpu.SemaphoreType.DMA,
        ],
    )


def _sc_adjacency_body(flat_hbm, val_hbm, mflat_hbm, a_out, am_out,
                       buf, fbuf0, fbuf1, vbuf0, vbuf1, mbuf, mfbuf,
                       sem0, sem1):
    wid = lax.axis_index("s") * 2 + lax.axis_index("c")
    ones = jnp.ones((16,), jnp.float32)
    zeros = jnp.zeros((16,), jnp.float32)
    izeros = jnp.zeros((16,), jnp.int32)
    fbase = wid * CWORDS
    fbufs, vbufs, sems = (fbuf0, fbuf1), (vbuf0, vbuf1), (sem0, sem1)
    nblk = E_ATOM // EBLK

    def _start(blk):
        slot = blk % 2
        return (
            pltpu.async_copy(flat_hbm.at[pl.ds(blk * EBLK, EBLK)],
                             fbufs[slot], sems[slot]),
            pltpu.async_copy(val_hbm.at[pl.ds(blk * EBLK, EBLK)],
                             vbufs[slot], sems[slot]),
        )

    pending = _start(0)

    def zbody(i, _):
        r = i >> 3
        cb = (i & 7) * 128
        for k in range(8):
            buf[r, pl.ds(cb + k * 16, 16)] = izeros
        return _
    lax.fori_loop(0, CWORDS // 128, zbody, 0)

    for blk in range(nblk):
        slot = blk % 2
        for h in pending:
            h.wait()
        if blk + 1 < nblk:
            pending = _start(blk + 1)
        fbuf, vbuf = fbufs[slot], vbufs[slot]

        def ebody(i, _):
            for k in range(4):
                off = i * 64 + k * 16
                rel = fbuf[pl.ds(off, 16)] - fbase
                m = rel.astype(jnp.uint32) < CWORDS
                plsc.addupdate_scatter(
                    buf, [rel >> 10, rel & (PCOL - 1)],
                    vbuf[pl.ds(off, 16)], mask=m)
            return _
        lax.fori_loop(0, EBLK // 64, ebody, 0)

    pltpu.sync_copy(buf, a_out.at[pl.ds(wid * CH, CH)])

    mwords = MCH * N_MOTIF
    mfbase = wid * mwords
    pltpu.sync_copy(mflat_hbm, mfbuf)

    def mzbody(i, _):
        r = i >> 2
        cb = (i & 3) * 128
        for k in range(8):
            mbuf[r, pl.ds(cb + k * 16, 16)] = zeros
        return _
    lax.fori_loop(0, mwords // 128, mzbody, 0)

    def mebody(i, _):
        for k in range(4):
            rel = mfbuf[pl.ds(i * 64 + k * 16, 16)] - mfbase
            m = rel.astype(jnp.uint32) < mwords
            plsc.addupdate_scatter(mbuf, [rel >> 9, rel & (N_MOTIF - 1)],
                                   ones, mask=m)
        return _
    lax.fori_loop(0, E_MOTIF // 64, mebody, 0)

    pltpu.sync_copy(mbuf, am_out.at[pl.ds(wid * MCH, MCH)])


def _dot(a, b, ca=1, cb=0):
    return lax.dot_general(a, b, (((ca,), (cb,)), ((), ())), precision=_PREC)


def _full(shape):
    return pl.BlockSpec(shape, lambda *_: (0,) * len(shape))


def _flat_body(e_ref, me_ref, f_ref, v_ref, mf_ref):
    src, dst = e_ref[0], e_ref[1]
    f_ref[...] = dst * PCOL + (src & (PCOL - 1))
    v_ref[...] = 1 << ((src >> 10) << 4)
    mf_ref[...] = me_ref[1] * N_MOTIF + me_ref[0]


def _flat_ids(edge_index, motif_edge_index):
    f, v, mf = pl.pallas_call(
        _flat_body,
        out_shape=[
            jax.ShapeDtypeStruct((E_ATOM // 128, 128), jnp.int32),
            jax.ShapeDtypeStruct((E_ATOM // 128, 128), jnp.int32),
            jax.ShapeDtypeStruct((E_MOTIF // 128, 128), jnp.int32),
        ],
    )(edge_index.reshape(2, E_ATOM // 128, 128),
      motif_edge_index.reshape(2, E_MOTIF // 128, 128))
    return f.reshape(E_ATOM), v.reshape(E_ATOM), mf.reshape(E_MOTIF)


def _gcn_pre_body(a_ref, xf_ref, aw_ref, ab_ref, w0_ref,
                  abf_ref, dinv_ref, hd_ref, hdb_ref):
    ap = a_ref[...]
    low = ap & 0xFFFF
    high = ap >> 16
    abf_ref[:, :PCOL] = low.astype(jnp.bfloat16)
    abf_ref[:, PCOL:] = high.astype(jnp.bfloat16)
    deg = (jnp.sum(low, axis=1, keepdims=True)
           + jnp.sum(high, axis=1, keepdims=True)).astype(jnp.float32) + 1.0
    dinv = lax.rsqrt(deg)
    dinv_ref[...] = dinv
    x0 = _dot(xf_ref[...], aw_ref[...]) + ab_ref[...]
    hd = dinv * _dot(x0, w0_ref[...])
    hd_ref[...] = hd
    hdb_ref[...] = hd.astype(jnp.bfloat16)


def _gcn_pre(A, atom_f, aw, ab, w0):
    return pl.pallas_call(
        _gcn_pre_body,
        grid=(NRB,),
        in_specs=[pl.BlockSpec((RB, PCOL), lambda i: (i, 0)),
                  pl.BlockSpec((RB, 128), lambda i: (i, 0)),
                  _full((128, HIDDEN)), _full((1, HIDDEN)),
                  _full((HIDDEN, HIDDEN))],
        out_specs=[pl.BlockSpec((RB, N_ATOM), lambda i: (i, 0)),
                   pl.BlockSpec((RB, 1), lambda i: (i, 0)),
                   pl.BlockSpec((RB, HIDDEN), lambda i: (i, 0)),
                   pl.BlockSpec((RB, HIDDEN), lambda i: (i, 0))],
        out_shape=[jax.ShapeDtypeStruct((N_ATOM, N_ATOM), jnp.bfloat16),
                   jax.ShapeDtypeStruct((N_ATOM, 1), jnp.float32),
                   jax.ShapeDtypeStruct((N_ATOM, HIDDEN), jnp.float32),
                   jax.ShapeDtypeStruct((N_ATOM, HIDDEN), jnp.bfloat16)],
    )(A, atom_f, aw, ab.reshape(1, HIDDEN), w0)


def _gcn_agg(abf_ref, hdbf_ref, hd_ref):
    t = lax.dot_general(abf_ref[...], hdbf_ref[...],
                        (((1,), (0,)), ((), ())),
                        preferred_element_type=jnp.float32)
    return t + hd_ref[...]


def _gcn_fused_mid(abf_ref, hdbf_ref, hd_ref, dinv_ref, b_ref,
                   wn_ref, o_ref, ob_ref):
    t = _gcn_agg(abf_ref, hdbf_ref, hd_ref)
    x = jnp.maximum(dinv_ref[...] * t + b_ref[...], 0.0)
    hd = dinv_ref[...] * _dot(x, wn_ref[...])
    o_ref[...] = hd
    ob_ref[...] = hd.astype(jnp.bfloat16)


def _gcn_fused_last(abf_ref, hdbf_ref, hd_ref, dinv_ref, b_ref, o_ref):
    t = _gcn_agg(abf_ref, hdbf_ref, hd_ref)
    o_ref[...] = jnp.maximum(dinv_ref[...] * t + b_ref[...], 0.0)


def _gcn_fused(Abf, hd, hdb, dinv, b, w_next):
    specs = [pl.BlockSpec((RB, N_ATOM), lambda i: (i, 0)),
             _full((N_ATOM, HIDDEN)),
             pl.BlockSpec((RB, HIDDEN), lambda i: (i, 0)),
             pl.BlockSpec((RB, 1), lambda i: (i, 0)),
             _full((1, HIDDEN))]
    args = [Abf, hdb, hd, dinv, b.reshape(1, HIDDEN)]
    blk = pl.BlockSpec((RB, HIDDEN), lambda i: (i, 0))
    if w_next is None:
        return pl.pallas_call(
            _gcn_fused_last,
            grid=(NRB,),
            in_specs=specs,
            out_specs=blk,
            out_shape=jax.ShapeDtypeStruct((N_ATOM, HIDDEN), jnp.float32),
        )(*args)
    specs.append(_full((HIDDEN, HIDDEN)))
    args.append(w_next)
    return pl.pallas_call(
        _gcn_fused_mid,
        grid=(NRB,),
        in_specs=specs,
        out_specs=[blk, blk],
        out_shape=[jax.ShapeDtypeStruct((N_ATOM, HIDDEN), jnp.float32),
                   jax.ShapeDtypeStruct((N_ATOM, HIDDEN), jnp.bfloat16)],
    )(*args)


def _softmax_exp(s):
    return jnp.exp((s - jnp.max(s, axis=1, keepdims=True))
                   .astype(jnp.bfloat16))


def _motif_body(am_ref, mf_ref, mw_ref, mb_ref,
                w1_ref, b1_ref, w2_ref, b2_ref,
                wqkv_ref, bqkv_ref, o_ref):
    m = _dot(mf_ref[...], mw_ref[...]) + mb_ref[...]
    Am = am_ref[...]
    for i in range(3):
        h = m + _dot(Am, m)
        h1 = jnp.maximum(_dot(h, w1_ref[i]) + b1_ref[i][None, :], 0.0)
        m = jnp.maximum(_dot(h1, w2_ref[i]) + b2_ref[i][None, :], 0.0)
    qkv = _dot(m, wqkv_ref[...], 1, 1) + bqkv_ref[...]
    outs = []
    scale = 1.0 / float(HD) ** 0.5
    for h in range(HEADS):
        q = qkv[:, h * HD:(h + 1) * HD] * scale
        k = qkv[:, HIDDEN + h * HD:HIDDEN + (h + 1) * HD]
        v = qkv[:, 2 * HIDDEN + h * HD:2 * HIDDEN + (h + 1) * HD]
        s = lax.dot_general(q, k, (((1,), (1,)), ((), ())),
                            precision=_PREC)
        e = _softmax_exp(s)
        vx = jnp.concatenate(
            [v, jnp.ones((N_MOTIF, 16), jnp.float32)], axis=1)
        ox = lax.dot_general(e, vx.astype(jnp.bfloat16),
                             (((1,), (0,)), ((), ())),
                             preferred_element_type=jnp.float32)
        o = ox[:, :HD] * (1.0 / ox[:, HD:HD + 1])
        outs.append(jnp.sum(o, axis=0, keepdims=True) * (1.0 / N_MOTIF))
    o_ref[...] = jnp.concatenate(outs, axis=1)


def _motif_pooled(Am, motif_f, mw, mb, w1, b1, w2, b2, wqkv, bqkv):
    return pl.pallas_call(
        _motif_body,
        out_shape=jax.ShapeDtypeStruct((1, HIDDEN), jnp.float32),
    )(Am, motif_f, mw, mb.reshape(1, HIDDEN), w1, b1, w2, b2,
      wqkv, bqkv.reshape(1, 3 * HIDDEN))


def _attn_pool_body(xb_ref, xf_ref, wq_ref, wk_ref, wv_ref,
                    bq_ref, bk_ref, bv_ref, o_ref, k_s, v_s):
    j = pl.program_id(1)
    L = xf_ref.shape[0]

    @pl.when(j == 0)
    def _():
        k_s[...] = _dot(xf_ref[...], wk_ref[...], 1, 1) + bk_ref[0]
        v = _dot(xf_ref[...], wv_ref[...], 1, 1) + bv_ref[0]
        v_s[:, :HD] = v.astype(jnp.bfloat16)
        v_s[:, HD:HD + 16] = jnp.ones((L, 16), jnp.bfloat16)

    q = (_dot(xb_ref[...], wq_ref[...], 1, 1) + bq_ref[0]) \
        * (1.0 / float(HD) ** 0.5)
    s = lax.dot_general(q, k_s[...], (((1,), (1,)), ((), ())),
                        precision=_PREC)
    e = _softmax_exp(s)
    ox = lax.dot_general(e, v_s[...], (((1,), (0,)), ((), ())),
                         preferred_element_type=jnp.float32)
    o = ox[:, :HD] * (1.0 / ox[:, HD:HD + 1])
    colsum = jnp.sum(o, axis=0, keepdims=True) * (1.0 / float(L))

    @pl.when(j == 0)
    def _():
        o_ref[...] = jnp.zeros_like(o_ref)

    o_ref[0] += colsum


def _attn_pool(x, wqkv, bqkv):
    L = x.shape[0]
    b3 = bqkv.reshape(3 * HEADS, 1, HD)
    return pl.pallas_call(
        _attn_pool_body,
        grid=(HEADS, L // RB),
        in_specs=[
            pl.BlockSpec((RB, HIDDEN), lambda h, j: (j, 0)),
            _full((L, HIDDEN)),
            pl.BlockSpec((HD, HIDDEN), lambda h, j: (h, 0)),
            pl.BlockSpec((HD, HIDDEN), lambda h, j: (HEADS + h, 0)),
            pl.BlockSpec((HD, HIDDEN), lambda h, j: (2 * HEADS + h, 0)),
            pl.BlockSpec((1, 1, HD), lambda h, j: (h, 0, 0)),
            pl.BlockSpec((1, 1, HD), lambda h, j: (HEADS + h, 0, 0)),
            pl.BlockSpec((1, 1, HD), lambda h, j: (2 * HEADS + h, 0, 0)),
        ],
        out_specs=pl.BlockSpec((1, 1, HD), lambda h, j: (h, 0, 0)),
        out_shape=jax.ShapeDtypeStruct((HEADS, 1, HD), jnp.float32),
        scratch_shapes=[pltpu.VMEM((L, HD), jnp.float32),
                        pltpu.VMEM((L, HD + 16), jnp.bfloat16)],
    )(x, x, wqkv, wqkv, wqkv, b3, b3, b3)


def _final_body(am_ref, mm_ref, wo_ref, bo_ref,
                w1_ref, b1_ref, w2_ref, b2_ref, o_ref):
    ag = _dot(am_ref[...], wo_ref[...], 1, 1) + bo_ref[...]
    mg = _dot(mm_ref[...], wo_ref[...], 1, 1) + bo_ref[...]
    c = jnp.concatenate([ag, mg], axis=1)
    h = jnp.maximum(_dot(c, w1_ref[...]) + b1_ref[...], 0.0)
    o_ref[...] = _dot(h, w2_ref[...]) + b2_ref[...]


def _final(atom_mean, motif_mean, wo, bo, w1, b1, w2, b2):
    return pl.pallas_call(
        _final_body,
        out_shape=jax.ShapeDtypeStruct((1, 128), jnp.float32),
    )(atom_mean, motif_mean, wo, bo.reshape(1, HIDDEN),
      w1, b1.reshape(1, HIDDEN), w2, b2.reshape(1, 128))


def _build_adjacency(edge_index, motif_edge_index):
    flat, vals, mflat = _flat_ids(edge_index, motif_edge_index)
    return _sc_adjacency_kernel()(flat, vals, mflat)


def kernel(atom_features, bond_features, motif_features, params,
           edge_index, motif_edge_index):
    del bond_features
    p = params
    A, Am = _build_adjacency(edge_index, motif_edge_index)

    Abf, dinv, hd, hdb = _gcn_pre(A, atom_features, p['atom_W'],
                                  p['atom_b'], p['gcn_W'][0])
    hd, hdb = _gcn_fused(Abf, hd, hdb, dinv, p['gcn_b'][0], p['gcn_W'][1])
    hd, hdb = _gcn_fused(Abf, hd, hdb, dinv, p['gcn_b'][1], p['gcn_W'][2])
    x = _gcn_fused(Abf, hd, hdb, dinv, p['gcn_b'][2], None)

    mm = _motif_pooled(Am, motif_features, p['motif_W'], p['motif_b'],
                       p['gin_W1'], p['gin_b1'], p['gin_W2'], p['gin_b2'],
                       p['attn_Wqkv'], p['attn_bqkv'])
    am = _attn_pool(x, p['attn_Wqkv'], p['attn_bqkv']).reshape(1, HIDDEN)

    latent = _final(am, mm, p['attn_Wo'], p['attn_bo'],
                    p['proj_W1'], p['proj_b1'], p['proj_W2'], p['proj_b2'])
    return latent.reshape(128)


# speedup vs baseline: 18.7922x; 1.0044x over previous
import functools

import jax
import jax.numpy as jnp
from jax import lax
from jax.experimental import pallas as pl
from jax.experimental.pallas import tpu as pltpu
from jax.experimental.pallas import tpu_sc as plsc

N_ATOM = 2048
E_ATOM = 65536
N_MOTIF = 512
E_MOTIF = 2048
HIDDEN = 256
HEADS = 4
HD = HIDDEN // HEADS

NW = 32
CH = 64
PCOL = N_ATOM // 2
CWORDS = CH * PCOL
EBLK = 4096
MCH = N_MOTIF // NW

RB = 512
NRB = N_ATOM // RB

_PREC = jax.lax.Precision.DEFAULT


@functools.cache
def _sc_adjacency_kernel():
    mesh = plsc.VectorSubcoreMesh(core_axis_name="c", subcore_axis_name="s")
    return pl.kernel(
        _sc_adjacency_body,
        mesh=mesh,
        compiler_params=pltpu.CompilerParams(needs_layout_passes=False),
        out_type=[
            jax.ShapeDtypeStruct((N_ATOM, PCOL), jnp.int32),
            jax.ShapeDtypeStruct((N_MOTIF, N_MOTIF), jnp.float32),
        ],
        scratch_types=[
            pltpu.VMEM((CH, PCOL), jnp.int32),
            pltpu.VMEM((EBLK,), jnp.int32),
            pltpu.VMEM((EBLK,), jnp.int32),
            pltpu.VMEM((EBLK,), jnp.int32),
            pltpu.VMEM((EBLK,), jnp.int32),
            pltpu.VMEM((MCH, N_MOTIF), jnp.float32),
            pltpu.VMEM((E_MOTIF,), jnp.int32),
            pltpu.SemaphoreType.DMA,
            pltpu.SemaphoreType.DMA,
        ],
    )


def _sc_adjacency_body(flat_hbm, val_hbm, mflat_hbm, a_out, am_out,
                       buf, fbuf0, fbuf1, vbuf0, vbuf1, mbuf, mfbuf,
                       sem0, sem1):
    wid = lax.axis_index("s") * 2 + lax.axis_index("c")
    ones = jnp.ones((16,), jnp.float32)
    zeros = jnp.zeros((16,), jnp.float32)
    izeros = jnp.zeros((16,), jnp.int32)
    fbase = wid * CWORDS
    fbufs, vbufs, sems = (fbuf0, fbuf1), (vbuf0, vbuf1), (sem0, sem1)
    nblk = E_ATOM // EBLK

    def _start(blk):
        slot = blk % 2
        return (
            pltpu.async_copy(flat_hbm.at[pl.ds(blk * EBLK, EBLK)],
                             fbufs[slot], sems[slot]),
            pltpu.async_copy(val_hbm.at[pl.ds(blk * EBLK, EBLK)],
                             vbufs[slot], sems[slot]),
        )

    pending = _start(0)

    def zbody(i, _):
        r = i >> 3
        cb = (i & 7) * 128
        for k in range(8):
            buf[r, pl.ds(cb + k * 16, 16)] = izeros
        return _
    lax.fori_loop(0, CWORDS // 128, zbody, 0)

    for blk in range(nblk):
        slot = blk % 2
        for h in pending:
            h.wait()
        if blk + 1 < nblk:
            pending = _start(blk + 1)
        fbuf, vbuf = fbufs[slot], vbufs[slot]

        def ebody(i, _):
            for k in range(8):
                off = i * 128 + k * 16
                rel = fbuf[pl.ds(off, 16)] - fbase
                m = rel.astype(jnp.uint32) < CWORDS
                plsc.addupdate_scatter(
                    buf, [rel >> 10, rel & (PCOL - 1)],
                    vbuf[pl.ds(off, 16)], mask=m)
            return _
        lax.fori_loop(0, EBLK // 128, ebody, 0)

    a_done = pltpu.async_copy(buf, a_out.at[pl.ds(wid * CH, CH)], sem0)

    mwords = MCH * N_MOTIF
    mfbase = wid * mwords
    pltpu.sync_copy(mflat_hbm, mfbuf)

    def mzbody(i, _):
        r = i >> 2
        cb = (i & 3) * 128
        for k in range(8):
            mbuf[r, pl.ds(cb + k * 16, 16)] = zeros
        return _
    lax.fori_loop(0, mwords // 128, mzbody, 0)

    def mebody(i, _):
        for k in range(4):
            rel = mfbuf[pl.ds(i * 64 + k * 16, 16)] - mfbase
            m = rel.astype(jnp.uint32) < mwords
            plsc.addupdate_scatter(mbuf, [rel >> 9, rel & (N_MOTIF - 1)],
                                   ones, mask=m)
        return _
    lax.fori_loop(0, E_MOTIF // 64, mebody, 0)

    pltpu.sync_copy(mbuf, am_out.at[pl.ds(wid * MCH, MCH)])
    a_done.wait()


def _dot(a, b, ca=1, cb=0):
    return lax.dot_general(a, b, (((ca,), (cb,)), ((), ())), precision=_PREC)


def _full(shape):
    return pl.BlockSpec(shape, lambda *_: (0,) * len(shape))


def _flat_body(e_ref, me_ref, f_ref, v_ref, mf_ref):
    src, dst = e_ref[0], e_ref[1]
    f_ref[...] = dst * PCOL + (src & (PCOL - 1))
    v_ref[...] = 1 << ((src >> 10) << 4)
    mf_ref[...] = me_ref[1] * N_MOTIF + me_ref[0]


def _flat_ids(edge_index, motif_edge_index):
    f, v, mf = pl.pallas_call(
        _flat_body,
        out_shape=[
            jax.ShapeDtypeStruct((E_ATOM // 128, 128), jnp.int32),
            jax.ShapeDtypeStruct((E_ATOM // 128, 128), jnp.int32),
            jax.ShapeDtypeStruct((E_MOTIF // 128, 128), jnp.int32),
        ],
    )(edge_index.reshape(2, E_ATOM // 128, 128),
      motif_edge_index.reshape(2, E_MOTIF // 128, 128))
    return f.reshape(E_ATOM), v.reshape(E_ATOM), mf.reshape(E_MOTIF)


def _gcn_pre_body(a_ref, xf_ref, aw_ref, ab_ref, w0_ref,
                  abf_ref, dinv_ref, hd_ref, hdb_ref):
    ap = a_ref[...]
    low = ap & 0xFFFF
    high = ap >> 16
    abf_ref[:, :PCOL] = low.astype(jnp.bfloat16)
    abf_ref[:, PCOL:] = high.astype(jnp.bfloat16)
    deg = (jnp.sum(low, axis=1, keepdims=True)
           + jnp.sum(high, axis=1, keepdims=True)).astype(jnp.float32) + 1.0
    dinv = lax.rsqrt(deg)
    dinv_ref[...] = dinv
    x0 = _dot(xf_ref[...], aw_ref[...]) + ab_ref[...]
    hd = dinv * _dot(x0, w0_ref[...])
    hd_ref[...] = hd
    hdb_ref[...] = hd.astype(jnp.bfloat16)


def _gcn_pre(A, atom_f, aw, ab, w0):
    return pl.pallas_call(
        _gcn_pre_body,
        grid=(NRB,),
        in_specs=[pl.BlockSpec((RB, PCOL), lambda i: (i, 0)),
                  pl.BlockSpec((RB, 128), lambda i: (i, 0)),
                  _full((128, HIDDEN)), _full((1, HIDDEN)),
                  _full((HIDDEN, HIDDEN))],
        out_specs=[pl.BlockSpec((RB, N_ATOM), lambda i: (i, 0)),
                   pl.BlockSpec((RB, 1), lambda i: (i, 0)),
                   pl.BlockSpec((RB, HIDDEN), lambda i: (i, 0)),
                   pl.BlockSpec((RB, HIDDEN), lambda i: (i, 0))],
        out_shape=[jax.ShapeDtypeStruct((N_ATOM, N_ATOM), jnp.bfloat16),
                   jax.ShapeDtypeStruct((N_ATOM, 1), jnp.float32),
                   jax.ShapeDtypeStruct((N_ATOM, HIDDEN), jnp.float32),
                   jax.ShapeDtypeStruct((N_ATOM, HIDDEN), jnp.bfloat16)],
    )(A, atom_f, aw, ab.reshape(1, HIDDEN), w0)


def _gcn_agg(abf_ref, hdbf_ref, hd_ref):
    t = lax.dot_general(abf_ref[...], hdbf_ref[...],
                        (((1,), (0,)), ((), ())),
                        preferred_element_type=jnp.float32)
    return t + hd_ref[...]


def _gcn_fused_mid(abf_ref, hdbf_ref, hd_ref, dinv_ref, b_ref,
                   wn_ref, o_ref, ob_ref):
    t = _gcn_agg(abf_ref, hdbf_ref, hd_ref)
    x = jnp.maximum(dinv_ref[...] * t + b_ref[...], 0.0)
    hd = dinv_ref[...] * _dot(x, wn_ref[...])
    o_ref[...] = hd
    ob_ref[...] = hd.astype(jnp.bfloat16)


def _gcn_fused_last(abf_ref, hdbf_ref, hd_ref, dinv_ref, b_ref, o_ref):
    t = _gcn_agg(abf_ref, hdbf_ref, hd_ref)
    o_ref[...] = jnp.maximum(dinv_ref[...] * t + b_ref[...], 0.0)


def _gcn_fused(Abf, hd, hdb, dinv, b, w_next):
    specs = [pl.BlockSpec((RB, N_ATOM), lambda i: (i, 0)),
             _full((N_ATOM, HIDDEN)),
             pl.BlockSpec((RB, HIDDEN), lambda i: (i, 0)),
             pl.BlockSpec((RB, 1), lambda i: (i, 0)),
             _full((1, HIDDEN))]
    args = [Abf, hdb, hd, dinv, b.reshape(1, HIDDEN)]
    blk = pl.BlockSpec((RB, HIDDEN), lambda i: (i, 0))
    if w_next is None:
        return pl.pallas_call(
            _gcn_fused_last,
            grid=(NRB,),
            in_specs=specs,
            out_specs=blk,
            out_shape=jax.ShapeDtypeStruct((N_ATOM, HIDDEN), jnp.float32),
        )(*args)
    specs.append(_full((HIDDEN, HIDDEN)))
    args.append(w_next)
    return pl.pallas_call(
        _gcn_fused_mid,
        grid=(NRB,),
        in_specs=specs,
        out_specs=[blk, blk],
        out_shape=[jax.ShapeDtypeStruct((N_ATOM, HIDDEN), jnp.float32),
                   jax.ShapeDtypeStruct((N_ATOM, HIDDEN), jnp.bfloat16)],
    )(*args)


def _softmax_exp(s):
    return jnp.exp((s - jnp.max(s, axis=1, keepdims=True))
                   .astype(jnp.bfloat16))


def _motif_body(am_ref, mf_ref, mw_ref, mb_ref,
                w1_ref, b1_ref, w2_ref, b2_ref,
                wqkv_ref, bqkv_ref, o_ref):
    m = _dot(mf_ref[...], mw_ref[...]) + mb_ref[...]
    Am = am_ref[...]
    for i in range(3):
        h = m + _dot(Am, m)
        h1 = jnp.maximum(_dot(h, w1_ref[i]) + b1_ref[i][None, :], 0.0)
        m = jnp.maximum(_dot(h1, w2_ref[i]) + b2_ref[i][None, :], 0.0)
    qkv = _dot(m, wqkv_ref[...], 1, 1) + bqkv_ref[...]
    outs = []
    scale = 1.0 / float(HD) ** 0.5
    for h in range(HEADS):
        q = qkv[:, h * HD:(h + 1) * HD] * scale
        k = qkv[:, HIDDEN + h * HD:HIDDEN + (h + 1) * HD]
        v = qkv[:, 2 * HIDDEN + h * HD:2 * HIDDEN + (h + 1) * HD]
        s = lax.dot_general(q, k, (((1,), (1,)), ((), ())),
                            precision=_PREC)
        e = _softmax_exp(s)
        vx = jnp.concatenate(
            [v, jnp.ones((N_MOTIF, 16), jnp.float32)], axis=1)
        ox = lax.dot_general(e, vx.astype(jnp.bfloat16),
                             (((1,), (0,)), ((), ())),
                             preferred_element_type=jnp.float32)
        o = ox[:, :HD] * (1.0 / ox[:, HD:HD + 1])
        outs.append(jnp.sum(o, axis=0, keepdims=True) * (1.0 / N_MOTIF))
    o_ref[...] = jnp.concatenate(outs, axis=1)


def _motif_pooled(Am, motif_f, mw, mb, w1, b1, w2, b2, wqkv, bqkv):
    return pl.pallas_call(
        _motif_body,
        out_shape=jax.ShapeDtypeStruct((1, HIDDEN), jnp.float32),
    )(Am, motif_f, mw, mb.reshape(1, HIDDEN), w1, b1, w2, b2,
      wqkv, bqkv.reshape(1, 3 * HIDDEN))


def _attn_pool_body(xb_ref, xf_ref, wq_ref, wk_ref, wv_ref,
                    bq_ref, bk_ref, bv_ref, o_ref, k_s, v_s):
    j = pl.program_id(1)
    L = xf_ref.shape[0]

    @pl.when(j == 0)
    def _():
        k_s[...] = _dot(xf_ref[...], wk_ref[...], 1, 1) + bk_ref[0]
        v = _dot(xf_ref[...], wv_ref[...], 1, 1) + bv_ref[0]
        v_s[:, :HD] = v.astype(jnp.bfloat16)
        v_s[:, HD:HD + 16] = jnp.ones((L, 16), jnp.bfloat16)

    q = (_dot(xb_ref[...], wq_ref[...], 1, 1) + bq_ref[0]) \
        * (1.0 / float(HD) ** 0.5)
    s = lax.dot_general(q, k_s[...], (((1,), (1,)), ((), ())),
                        precision=_PREC)
    e = _softmax_exp(s)
    ox = lax.dot_general(e, v_s[...], (((1,), (0,)), ((), ())),
                         preferred_element_type=jnp.float32)
    o = ox[:, :HD] * (1.0 / ox[:, HD:HD + 1])
    colsum = jnp.sum(o, axis=0, keepdims=True) * (1.0 / float(L))

    @pl.when(j == 0)
    def _():
        o_ref[...] = jnp.zeros_like(o_ref)

    o_ref[0] += colsum


def _attn_pool(x, wqkv, bqkv):
    L = x.shape[0]
    b3 = bqkv.reshape(3 * HEADS, 1, HD)
    return pl.pallas_call(
        _attn_pool_body,
        grid=(HEADS, L // RB),
        in_specs=[
            pl.BlockSpec((RB, HIDDEN), lambda h, j: (j, 0)),
            _full((L, HIDDEN)),
            pl.BlockSpec((HD, HIDDEN), lambda h, j: (h, 0)),
            pl.BlockSpec((HD, HIDDEN), lambda h, j: (HEADS + h, 0)),
            pl.BlockSpec((HD, HIDDEN), lambda h, j: (2 * HEADS + h, 0)),
            pl.BlockSpec((1, 1, HD), lambda h, j: (h, 0, 0)),
            pl.BlockSpec((1, 1, HD), lambda h, j: (HEADS + h, 0, 0)),
            pl.BlockSpec((1, 1, HD), lambda h, j: (2 * HEADS + h, 0, 0)),
        ],
        out_specs=pl.BlockSpec((1, 1, HD), lambda h, j: (h, 0, 0)),
        out_shape=jax.ShapeDtypeStruct((HEADS, 1, HD), jnp.float32),
        scratch_shapes=[pltpu.VMEM((L, HD), jnp.float32),
                        pltpu.VMEM((L, HD + 16), jnp.bfloat16)],
    )(x, x, wqkv, wqkv, wqkv, b3, b3, b3)


def _final_body(am_ref, mm_ref, wo_ref, bo_ref,
                w1_ref, b1_ref, w2_ref, b2_ref, o_ref):
    ag = _dot(am_ref[...], wo_ref[...], 1, 1) + bo_ref[...]
    mg = _dot(mm_ref[...], wo_ref[...], 1, 1) + bo_ref[...]
    c = jnp.concatenate([ag, mg], axis=1)
    h = jnp.maximum(_dot(c, w1_ref[...]) + b1_ref[...], 0.0)
    o_ref[...] = _dot(h, w2_ref[...]) + b2_ref[...]


def _final(atom_mean, motif_mean, wo, bo, w1, b1, w2, b2):
    return pl.pallas_call(
        _final_body,
        out_shape=jax.ShapeDtypeStruct((1, 128), jnp.float32),
    )(atom_mean, motif_mean, wo, bo.reshape(1, HIDDEN),
      w1, b1.reshape(1, HIDDEN), w2, b2.reshape(1, 128))


def _build_adjacency(edge_index, motif_edge_index):
    flat, vals, mflat = _flat_ids(edge_index, motif_edge_index)
    return _sc_adjacency_kernel()(flat, vals, mflat)


def kernel(atom_features, bond_features, motif_features, params,
           edge_index, motif_edge_index):
    del bond_features
    p = params
    A, Am = _build_adjacency(edge_index, motif_edge_index)

    Abf, dinv, hd, hdb = _gcn_pre(A, atom_features, p['atom_W'],
                                  p['atom_b'], p['gcn_W'][0])
    hd, hdb = _gcn_fused(Abf, hd, hdb, dinv, p['gcn_b'][0], p['gcn_W'][1])
    hd, hdb = _gcn_fused(Abf, hd, hdb, dinv, p['gcn_b'][1], p['gcn_W'][2])
    x = _gcn_fused(Abf, hd, hdb, dinv, p['gcn_b'][2], None)

    mm = _motif_pooled(Am, motif_features, p['motif_W'], p['motif_b'],
                       p['gin_W1'], p['gin_b1'], p['gin_W2'], p['gin_b2'],
                       p['attn_Wqkv'], p['attn_bqkv'])
    am = _attn_pool(x, p['attn_Wqkv'], p['attn_bqkv']).reshape(1, HIDDEN)

    latent = _final(am, mm, p['attn_Wo'], p['attn_bo'],
                    p['proj_W1'], p['proj_b1'], p['proj_W2'], p['proj_b2'])
    return latent.reshape(128)


# speedup vs baseline: 18.9438x; 1.0081x over previous
import functools

import jax
import jax.numpy as jnp
from jax import lax
from jax.experimental import pallas as pl
from jax.experimental.pallas import tpu as pltpu
from jax.experimental.pallas import tpu_sc as plsc

N_ATOM = 2048
E_ATOM = 65536
N_MOTIF = 512
E_MOTIF = 2048
HIDDEN = 256
HEADS = 4
HD = HIDDEN // HEADS

NW = 32
CH = 64
PCOL = N_ATOM // 2
CWORDS = CH * PCOL
EBLK = 4096
MCH = N_MOTIF // NW

RB = 512
NRB = N_ATOM // RB

_PREC = jax.lax.Precision.DEFAULT


@functools.cache
def _sc_atom_kernel():
    mesh = plsc.VectorSubcoreMesh(core_axis_name="c", subcore_axis_name="s")
    return pl.kernel(
        _sc_atom_body,
        mesh=mesh,
        compiler_params=pltpu.CompilerParams(needs_layout_passes=False),
        out_type=jax.ShapeDtypeStruct((N_ATOM, PCOL), jnp.int32),
        scratch_types=[
            pltpu.VMEM((CH, PCOL), jnp.int32),
            pltpu.VMEM((EBLK,), jnp.int32),
            pltpu.VMEM((EBLK,), jnp.int32),
            pltpu.VMEM((EBLK,), jnp.int32),
            pltpu.VMEM((EBLK,), jnp.int32),
            pltpu.SemaphoreType.DMA,
            pltpu.SemaphoreType.DMA,
        ],
    )


@functools.cache
def _sc_motif_kernel():
    mesh = plsc.VectorSubcoreMesh(core_axis_name="c", subcore_axis_name="s")
    return pl.kernel(
        _sc_motif_body,
        mesh=mesh,
        compiler_params=pltpu.CompilerParams(needs_layout_passes=False),
        out_type=jax.ShapeDtypeStruct((N_MOTIF, N_MOTIF), jnp.float32),
        scratch_types=[
            pltpu.VMEM((MCH, N_MOTIF), jnp.float32),
            pltpu.VMEM((E_MOTIF,), jnp.int32),
        ],
    )


def _sc_atom_body(flat_hbm, val_hbm, a_out,
                  buf, fbuf0, fbuf1, vbuf0, vbuf1, sem0, sem1):
    wid = lax.axis_index("s") * 2 + lax.axis_index("c")
    izeros = jnp.zeros((16,), jnp.int32)
    fbase = wid * CWORDS
    fbufs, vbufs, sems = (fbuf0, fbuf1), (vbuf0, vbuf1), (sem0, sem1)
    nblk = E_ATOM // EBLK

    def _start(blk):
        slot = blk % 2
        return (
            pltpu.async_copy(flat_hbm.at[pl.ds(blk * EBLK, EBLK)],
                             fbufs[slot], sems[slot]),
            pltpu.async_copy(val_hbm.at[pl.ds(blk * EBLK, EBLK)],
                             vbufs[slot], sems[slot]),
        )

    pending = _start(0)

    def zbody(i, _):
        r = i >> 3
        cb = (i & 7) * 128
        for k in range(8):
            buf[r, pl.ds(cb + k * 16, 16)] = izeros
        return _
    lax.fori_loop(0, CWORDS // 128, zbody, 0)

    for blk in range(nblk):
        slot = blk % 2
        for h in pending:
            h.wait()
        if blk + 1 < nblk:
            pending = _start(blk + 1)
        fbuf, vbuf = fbufs[slot], vbufs[slot]

        def ebody(i, _):
            for k in range(8):
                off = i * 128 + k * 16
                rel = fbuf[pl.ds(off, 16)] - fbase
                m = rel.astype(jnp.uint32) < CWORDS
                plsc.addupdate_scatter(
                    buf, [rel >> 10, rel & (PCOL - 1)],
                    vbuf[pl.ds(off, 16)], mask=m)
            return _
        lax.fori_loop(0, EBLK // 128, ebody, 0)

    pltpu.sync_copy(buf, a_out.at[pl.ds(wid * CH, CH)])


def _sc_motif_body(mflat_hbm, am_out, mbuf, mfbuf):
    wid = lax.axis_index("s") * 2 + lax.axis_index("c")
    ones = jnp.ones((16,), jnp.float32)
    zeros = jnp.zeros((16,), jnp.float32)
    mwords = MCH * N_MOTIF
    mfbase = wid * mwords
    pltpu.sync_copy(mflat_hbm, mfbuf)

    def mzbody(i, _):
        r = i >> 2
        cb = (i & 3) * 128
        for k in range(8):
            mbuf[r, pl.ds(cb + k * 16, 16)] = zeros
        return _
    lax.fori_loop(0, mwords // 128, mzbody, 0)

    def mebody(i, _):
        for k in range(4):
            rel = mfbuf[pl.ds(i * 64 + k * 16, 16)] - mfbase
            m = rel.astype(jnp.uint32) < mwords
            plsc.addupdate_scatter(mbuf, [rel >> 9, rel & (N_MOTIF - 1)],
                                   ones, mask=m)
        return _
    lax.fori_loop(0, E_MOTIF // 64, mebody, 0)

    pltpu.sync_copy(mbuf, am_out.at[pl.ds(wid * MCH, MCH)])


def _dot(a, b, ca=1, cb=0):
    return lax.dot_general(a, b, (((ca,), (cb,)), ((), ())), precision=_PREC)


def _full(shape):
    return pl.BlockSpec(shape, lambda *_: (0,) * len(shape))


def _flat_body(e_ref, me_ref, f_ref, v_ref, mf_ref):
    src, dst = e_ref[0], e_ref[1]
    f_ref[...] = dst * PCOL + (src & (PCOL - 1))
    v_ref[...] = 1 << ((src >> 10) << 4)
    mf_ref[...] = me_ref[1] * N_MOTIF + me_ref[0]


def _flat_ids(edge_index, motif_edge_index):
    f, v, mf = pl.pallas_call(
        _flat_body,
        out_shape=[
            jax.ShapeDtypeStruct((E_ATOM // 128, 128), jnp.int32),
            jax.ShapeDtypeStruct((E_ATOM // 128, 128), jnp.int32),
            jax.ShapeDtypeStruct((E_MOTIF // 128, 128), jnp.int32),
        ],
    )(edge_index.reshape(2, E_ATOM // 128, 128),
      motif_edge_index.reshape(2, E_MOTIF // 128, 128))
    return f.reshape(E_ATOM), v.reshape(E_ATOM), mf.reshape(E_MOTIF)


def _gcn_pre_body(a_ref, xf_ref, aw_ref, ab_ref, w0_ref,
                  abf_ref, dinv_ref, hd_ref, hdb_ref):
    ap = a_ref[...]
    low = ap & 0xFFFF
    high = ap >> 16
    abf_ref[:, :PCOL] = low.astype(jnp.bfloat16)
    abf_ref[:, PCOL:] = high.astype(jnp.bfloat16)
    deg = (jnp.sum(low, axis=1, keepdims=True)
           + jnp.sum(high, axis=1, keepdims=True)).astype(jnp.float32) + 1.0
    dinv = lax.rsqrt(deg)
    dinv_ref[...] = dinv
    x0 = _dot(xf_ref[...], aw_ref[...]) + ab_ref[...]
    hd = dinv * _dot(x0, w0_ref[...])
    hd_ref[...] = hd
    hdb_ref[...] = hd.astype(jnp.bfloat16)


def _gcn_pre(A, atom_f, aw, ab, w0):
    return pl.pallas_call(
        _gcn_pre_body,
        grid=(NRB,),
        in_specs=[pl.BlockSpec((RB, PCOL), lambda i: (i, 0)),
                  pl.BlockSpec((RB, 128), lambda i: (i, 0)),
                  _full((128, HIDDEN)), _full((1, HIDDEN)),
                  _full((HIDDEN, HIDDEN))],
        out_specs=[pl.BlockSpec((RB, N_ATOM), lambda i: (i, 0)),
                   pl.BlockSpec((RB, 1), lambda i: (i, 0)),
                   pl.BlockSpec((RB, HIDDEN), lambda i: (i, 0)),
                   pl.BlockSpec((RB, HIDDEN), lambda i: (i, 0))],
        out_shape=[jax.ShapeDtypeStruct((N_ATOM, N_ATOM), jnp.bfloat16),
                   jax.ShapeDtypeStruct((N_ATOM, 1), jnp.float32),
                   jax.ShapeDtypeStruct((N_ATOM, HIDDEN), jnp.float32),
                   jax.ShapeDtypeStruct((N_ATOM, HIDDEN), jnp.bfloat16)],
    )(A, atom_f, aw, ab.reshape(1, HIDDEN), w0)


def _gcn_agg(abf_ref, hdbf_ref, hd_ref):
    t = lax.dot_general(abf_ref[...], hdbf_ref[...],
                        (((1,), (0,)), ((), ())),
                        preferred_element_type=jnp.float32)
    return t + hd_ref[...]


def _gcn_fused_mid(abf_ref, hdbf_ref, hd_ref, dinv_ref, b_ref,
                   wn_ref, o_ref, ob_ref):
    t = _gcn_agg(abf_ref, hdbf_ref, hd_ref)
    x = jnp.maximum(dinv_ref[...] * t + b_ref[...], 0.0)
    hd = dinv_ref[...] * _dot(x, wn_ref[...])
    o_ref[...] = hd
    ob_ref[...] = hd.astype(jnp.bfloat16)


def _gcn_fused_last(abf_ref, hdbf_ref, hd_ref, dinv_ref, b_ref, o_ref):
    t = _gcn_agg(abf_ref, hdbf_ref, hd_ref)
    o_ref[...] = jnp.maximum(dinv_ref[...] * t + b_ref[...], 0.0)


def _gcn_fused(Abf, hd, hdb, dinv, b, w_next):
    specs = [pl.BlockSpec((RB, N_ATOM), lambda i: (i, 0)),
             _full((N_ATOM, HIDDEN)),
             pl.BlockSpec((RB, HIDDEN), lambda i: (i, 0)),
             pl.BlockSpec((RB, 1), lambda i: (i, 0)),
             _full((1, HIDDEN))]
    args = [Abf, hdb, hd, dinv, b.reshape(1, HIDDEN)]
    blk = pl.BlockSpec((RB, HIDDEN), lambda i: (i, 0))
    if w_next is None:
        return pl.pallas_call(
            _gcn_fused_last,
            grid=(NRB,),
            in_specs=specs,
            out_specs=blk,
            out_shape=jax.ShapeDtypeStruct((N_ATOM, HIDDEN), jnp.float32),
        )(*args)
    specs.append(_full((HIDDEN, HIDDEN)))
    args.append(w_next)
    return pl.pallas_call(
        _gcn_fused_mid,
        grid=(NRB,),
        in_specs=specs,
        out_specs=[blk, blk],
        out_shape=[jax.ShapeDtypeStruct((N_ATOM, HIDDEN), jnp.float32),
                   jax.ShapeDtypeStruct((N_ATOM, HIDDEN), jnp.bfloat16)],
    )(*args)


def _softmax_exp(s):
    return jnp.exp((s - jnp.max(s, axis=1, keepdims=True))
                   .astype(jnp.bfloat16))


def _motif_body(am_ref, mf_ref, mw_ref, mb_ref,
                w1_ref, b1_ref, w2_ref, b2_ref,
                wqkv_ref, bqkv_ref, o_ref):
    m = _dot(mf_ref[...], mw_ref[...]) + mb_ref[...]
    Am = am_ref[...]
    for i in range(3):
        h = m + _dot(Am, m)
        h1 = jnp.maximum(_dot(h, w1_ref[i]) + b1_ref[i][None, :], 0.0)
        m = jnp.maximum(_dot(h1, w2_ref[i]) + b2_ref[i][None, :], 0.0)
    qkv = _dot(m, wqkv_ref[...], 1, 1) + bqkv_ref[...]
    outs = []
    scale = 1.0 / float(HD) ** 0.5
    for h in range(HEADS):
        q = qkv[:, h * HD:(h + 1) * HD] * scale
        k = qkv[:, HIDDEN + h * HD:HIDDEN + (h + 1) * HD]
        v = qkv[:, 2 * HIDDEN + h * HD:2 * HIDDEN + (h + 1) * HD]
        s = lax.dot_general(q, k, (((1,), (1,)), ((), ())),
                            precision=_PREC)
        e = _softmax_exp(s)
        vx = jnp.concatenate(
            [v, jnp.ones((N_MOTIF, 16), jnp.float32)], axis=1)
        ox = lax.dot_general(e, vx.astype(jnp.bfloat16),
                             (((1,), (0,)), ((), ())),
                             preferred_element_type=jnp.float32)
        o = ox[:, :HD] * (1.0 / ox[:, HD:HD + 1])
        outs.append(jnp.sum(o, axis=0, keepdims=True) * (1.0 / N_MOTIF))
    o_ref[...] = jnp.concatenate(outs, axis=1)


def _motif_pooled(Am, motif_f, mw, mb, w1, b1, w2, b2, wqkv, bqkv):
    return pl.pallas_call(
        _motif_body,
        out_shape=jax.ShapeDtypeStruct((1, HIDDEN), jnp.float32),
    )(Am, motif_f, mw, mb.reshape(1, HIDDEN), w1, b1, w2, b2,
      wqkv, bqkv.reshape(1, 3 * HIDDEN))


def _attn_pool_body(xb_ref, xf_ref, wq_ref, wk_ref, wv_ref,
                    bq_ref, bk_ref, bv_ref, o_ref, k_s, v_s):
    j = pl.program_id(1)
    L = xf_ref.shape[0]

    @pl.when(j == 0)
    def _():
        k_s[...] = _dot(xf_ref[...], wk_ref[...], 1, 1) + bk_ref[0]
        v = _dot(xf_ref[...], wv_ref[...], 1, 1) + bv_ref[0]
        v_s[:, :HD] = v.astype(jnp.bfloat16)
        v_s[:, HD:HD + 16] = jnp.ones((L, 16), jnp.bfloat16)

    q = (_dot(xb_ref[...], wq_ref[...], 1, 1) + bq_ref[0]) \
        * (1.0 / float(HD) ** 0.5)
    s = lax.dot_general(q, k_s[...], (((1,), (1,)), ((), ())),
                        precision=_PREC)
    e = _softmax_exp(s)
    ox = lax.dot_general(e, v_s[...], (((1,), (0,)), ((), ())),
                         preferred_element_type=jnp.float32)
    o = ox[:, :HD] * (1.0 / ox[:, HD:HD + 1])
    colsum = jnp.sum(o, axis=0, keepdims=True) * (1.0 / float(L))

    @pl.when(j == 0)
    def _():
        o_ref[...] = jnp.zeros_like(o_ref)

    o_ref[0] += colsum


def _attn_pool(x, wqkv, bqkv):
    L = x.shape[0]
    b3 = bqkv.reshape(3 * HEADS, 1, HD)
    return pl.pallas_call(
        _attn_pool_body,
        grid=(HEADS, L // RB),
        in_specs=[
            pl.BlockSpec((RB, HIDDEN), lambda h, j: (j, 0)),
            _full((L, HIDDEN)),
            pl.BlockSpec((HD, HIDDEN), lambda h, j: (h, 0)),
            pl.BlockSpec((HD, HIDDEN), lambda h, j: (HEADS + h, 0)),
            pl.BlockSpec((HD, HIDDEN), lambda h, j: (2 * HEADS + h, 0)),
            pl.BlockSpec((1, 1, HD), lambda h, j: (h, 0, 0)),
            pl.BlockSpec((1, 1, HD), lambda h, j: (HEADS + h, 0, 0)),
            pl.BlockSpec((1, 1, HD), lambda h, j: (2 * HEADS + h, 0, 0)),
        ],
        out_specs=pl.BlockSpec((1, 1, HD), lambda h, j: (h, 0, 0)),
        out_shape=jax.ShapeDtypeStruct((HEADS, 1, HD), jnp.float32),
        scratch_shapes=[pltpu.VMEM((L, HD), jnp.float32),
                        pltpu.VMEM((L, HD + 16), jnp.bfloat16)],
    )(x, x, wqkv, wqkv, wqkv, b3, b3, b3)


def _final_body(am_ref, mm_ref, wo_ref, bo_ref,
                w1_ref, b1_ref, w2_ref, b2_ref, o_ref):
    ag = _dot(am_ref[...], wo_ref[...], 1, 1) + bo_ref[...]
    mg = _dot(mm_ref[...], wo_ref[...], 1, 1) + bo_ref[...]
    c = jnp.concatenate([ag, mg], axis=1)
    h = jnp.maximum(_dot(c, w1_ref[...]) + b1_ref[...], 0.0)
    o_ref[...] = _dot(h, w2_ref[...]) + b2_ref[...]


def _final(atom_mean, motif_mean, wo, bo, w1, b1, w2, b2):
    return pl.pallas_call(
        _final_body,
        out_shape=jax.ShapeDtypeStruct((1, 128), jnp.float32),
    )(atom_mean, motif_mean, wo, bo.reshape(1, HIDDEN),
      w1, b1.reshape(1, HIDDEN), w2, b2.reshape(1, 128))


def _build_adjacency(edge_index, motif_edge_index):
    flat, vals, mflat = _flat_ids(edge_index, motif_edge_index)
    Am = _sc_motif_kernel()(mflat)
    A = _sc_atom_kernel()(flat, vals)
    return A, Am


def kernel(atom_features, bond_features, motif_features, params,
           edge_index, motif_edge_index):
    del bond_features
    p = params
    A, Am = _build_adjacency(edge_index, motif_edge_index)

    mm = _motif_pooled(Am, motif_features, p['motif_W'], p['motif_b'],
                       p['gin_W1'], p['gin_b1'], p['gin_W2'], p['gin_b2'],
                       p['attn_Wqkv'], p['attn_bqkv'])

    Abf, dinv, hd, hdb = _gcn_pre(A, atom_features, p['atom_W'],
                                  p['atom_b'], p['gcn_W'][0])
    hd, hdb = _gcn_fused(Abf, hd, hdb, dinv, p['gcn_b'][0], p['gcn_W'][1])
    hd, hdb = _gcn_fused(Abf, hd, hdb, dinv, p['gcn_b'][1], p['gcn_W'][2])
    x = _gcn_fused(Abf, hd, hdb, dinv, p['gcn_b'][2], None)

    am = _attn_pool(x, p['attn_Wqkv'], p['attn_bqkv']).reshape(1, HIDDEN)

    latent = _final(am, mm, p['attn_Wo'], p['attn_bo'],
                    p['proj_W1'], p['proj_b1'], p['proj_W2'], p['proj_b2'])
    return latent.reshape(128)
